# baseline (speedup 1.0000x reference)
import functools

import jax
import jax.numpy as jnp
from jax import lax
from jax.experimental import pallas as pl
from jax.experimental.pallas import tpu as pltpu
from jax.experimental.pallas import tpu_sc as plsc

S = 2048
H = 1024
NH = 16
DH = 64
E = 8
FF = 4096
CAP = 256
EPS = 1e-6
TBLK = 256
QBLK = 512
FBLK = 512


def _ln(x, w, b):
    mu = jnp.mean(x, axis=-1, keepdims=True)
    var = jnp.mean((x - mu) ** 2, axis=-1, keepdims=True)
    return (x - mu) / jnp.sqrt(var + EPS) * w + b


def _qkv_body(x_ref, nw_ref, nb_ref, w_ref, b_ref, o_ref):
    h = _ln(x_ref[...], nw_ref[...], nb_ref[...])
    o_ref[...] = (
        jnp.dot(h, w_ref[...], preferred_element_type=jnp.float32) + b_ref[...]
    )


def _qkv_call(x, nw, nb, w, b):
    return pl.pallas_call(
        _qkv_body,
        grid=(3, S // TBLK),
        in_specs=[
            pl.BlockSpec((TBLK, H), lambda j, i: (i, 0)),
            pl.BlockSpec((1, H), lambda j, i: (0, 0)),
            pl.BlockSpec((1, H), lambda j, i: (0, 0)),
            pl.BlockSpec((H, H), lambda j, i: (0, j)),
            pl.BlockSpec((1, H), lambda j, i: (0, j)),
        ],
        out_specs=pl.BlockSpec((TBLK, H), lambda j, i: (i, j)),
        out_shape=jax.ShapeDtypeStruct((S, 3 * H), jnp.float32),
    )(x, nw, nb, w, b)


def _attn_body(q_ref, k_ref, v_ref, o_ref):
    qb = pl.program_id(1)
    q = q_ref[0]
    q_idx = qb * QBLK + lax.broadcasted_iota(jnp.int32, (QBLK, QBLK), 0)
    k_iota = lax.broadcasted_iota(jnp.int32, (QBLK, QBLK), 1)

    def body(kb, carry):
        m, l, acc = carry
        k = k_ref[0, pl.ds(kb * QBLK, QBLK), :]
        v = v_ref[0, pl.ds(kb * QBLK, QBLK), :]
        s = lax.dot_general(
            q, k, (((1,), (1,)), ((), ())), preferred_element_type=jnp.float32
        ) * 0.125
        s = jnp.where(kb * QBLK + k_iota <= q_idx, s, jnp.float32(-1e9))
        m_new = jnp.maximum(m, jnp.max(s, axis=-1, keepdims=True))
        p = jnp.exp(s - m_new)
        alpha = jnp.exp(m - m_new)
        l_new = l * alpha + jnp.sum(p, axis=-1, keepdims=True)
        acc_new = acc * alpha + jnp.dot(
            p, v, preferred_element_type=jnp.float32
        )
        return m_new, l_new, acc_new

    init = (
        jnp.full((QBLK, 1), -1e30, jnp.float32),
        jnp.zeros((QBLK, 1), jnp.float32),
        jnp.zeros((QBLK, DH), jnp.float32),
    )
    m, l, acc = lax.fori_loop(0, qb + 1, body, init)
    o_ref[0] = acc / l


def _attn_call(q3, k3, v3):
    return pl.pallas_call(
        _attn_body,
        grid=(NH, S // QBLK),
        in_specs=[
            pl.BlockSpec((1, QBLK, DH), lambda h, i: (h, i, 0)),
            pl.BlockSpec((1, S, DH), lambda h, i: (h, 0, 0)),
            pl.BlockSpec((1, S, DH), lambda h, i: (h, 0, 0)),
        ],
        out_specs=pl.BlockSpec((1, QBLK, DH), lambda h, i: (h, i, 0)),
        out_shape=jax.ShapeDtypeStruct((NH, S, DH), jnp.float32),
    )(q3, k3, v3)


def _post_body(ctx_ref, x_ref, ow_ref, ob_ref, nw_ref, nb_ref, gw_ref,
               res_ref, hm_ref, lg_ref):
    attn_out = (
        jnp.dot(ctx_ref[...], ow_ref[...], preferred_element_type=jnp.float32)
        + ob_ref[...]
    )
    res = x_ref[...] + attn_out
    res_ref[...] = res
    hm = _ln(res, nw_ref[...], nb_ref[...])
    hm_ref[...] = hm
    lg_ref[...] = jnp.dot(hm, gw_ref[...], preferred_element_type=jnp.float32)


def _post_call(ctx, x, ow, ob, nw, nb, gwp):
    return pl.pallas_call(
        _post_body,
        grid=(S // TBLK,),
        in_specs=[
            pl.BlockSpec((TBLK, H), lambda i: (i, 0)),
            pl.BlockSpec((TBLK, H), lambda i: (i, 0)),
            pl.BlockSpec((H, H), lambda i: (0, 0)),
            pl.BlockSpec((1, H), lambda i: (0, 0)),
            pl.BlockSpec((1, H), lambda i: (0, 0)),
            pl.BlockSpec((1, H), lambda i: (0, 0)),
            pl.BlockSpec((H, 128), lambda i: (0, 0)),
        ],
        out_specs=[
            pl.BlockSpec((TBLK, H), lambda i: (i, 0)),
            pl.BlockSpec((TBLK, H), lambda i: (i, 0)),
            pl.BlockSpec((TBLK, 128), lambda i: (i, 0)),
        ],
        out_shape=[
            jax.ShapeDtypeStruct((S, H), jnp.float32),
            jax.ShapeDtypeStruct((S, H), jnp.float32),
            jax.ShapeDtypeStruct((S, 128), jnp.float32),
        ],
    )(ctx, x, ow, ob, nw, nb, gwp)


def _route_body(lg_ref, g1_ref, cidx_ref, didx_ref, slotk_ref):
    lane = lax.broadcasted_iota(jnp.int32, (TBLK, 128), 1)
    r = lax.broadcasted_iota(jnp.int32, (TBLK, TBLK), 0)
    c = lax.broadcasted_iota(jnp.int32, (TBLK, TBLK), 1)
    tri = (c < r).astype(jnp.float32)
    counts = jnp.zeros((1, 128), jnp.float32)
    for b in range(S // TBLK):
        sl = pl.ds(b * TBLK, TBLK)
        lg = jnp.where(lane < E, lg_ref[sl, :], jnp.float32(-1e30))
        m = jnp.max(lg, axis=1, keepdims=True)
        ex = jnp.exp(lg - m)
        g = ex / jnp.sum(ex, axis=1, keepdims=True)
        gm = jnp.max(g, axis=1, keepdims=True)
        is_max = jnp.logical_and(g == gm, lane < E)
        eidx = jnp.min(jnp.where(is_max, lane, 128), axis=1, keepdims=True)
        mask1 = (lane == eidx).astype(jnp.float32)
        loc_excl = (
            jnp.dot(tri, mask1, preferred_element_type=jnp.float32) + counts
        )
        counts = counts + jnp.sum(mask1, axis=0, keepdims=True)
        loc1 = jnp.sum(loc_excl * mask1, axis=1, keepdims=True)
        keep = loc1 < jnp.float32(CAP)
        g1 = jnp.sum(g * mask1, axis=1, keepdims=True)
        g1_ref[sl, :] = jnp.where(keep, g1, 0.0)
        slot = eidx * CAP + loc1.astype(jnp.int32)
        cidx_ref[sl, :] = jnp.where(keep, slot, 0)
        slotk_ref[sl, :] = jnp.where(keep, slot, -1)
    slotk = slotk_ref[...]
    for sb in range(S // TBLK):
        s_iota = sb * TBLK + lax.broadcasted_iota(jnp.int32, (S, TBLK), 1)
        eq = slotk == s_iota
        t_col = lax.broadcasted_iota(jnp.int32, (S, TBLK), 0)
        inv = jnp.sum(jnp.where(eq, t_col, 0), axis=0, keepdims=True)
        didx_ref[:, pl.ds(sb * TBLK, TBLK)] = inv


def _route_call(logits):
    return pl.pallas_call(
        _route_body,
        grid=(1,),
        in_specs=[pl.BlockSpec((S, 128), lambda i: (0, 0))],
        out_specs=[
            pl.BlockSpec((S, 1), lambda i: (0, 0)),
            pl.BlockSpec((S, 1), lambda i: (0, 0)),
            pl.BlockSpec((1, S), lambda i: (0, 0)),
        ],
        out_shape=[
            jax.ShapeDtypeStruct((S, 1), jnp.float32),
            jax.ShapeDtypeStruct((S, 1), jnp.int32),
            jax.ShapeDtypeStruct((1, S), jnp.int32),
        ],
        scratch_shapes=[pltpu.VMEM((S, 1), jnp.int32)],
    )(logits)


def _sc_gather(table, idx):
    info = plsc.get_sparse_core_info()
    nw = info.num_cores * info.num_subcores
    b = idx.shape[0]
    d = table.shape[1]
    b_per_w = b // nw
    mesh = plsc.VectorSubcoreMesh(core_axis_name="c", subcore_axis_name="s")

    @functools.partial(
        pl.kernel,
        mesh=mesh,
        out_type=jax.ShapeDtypeStruct((b, d), jnp.float32),
        scratch_types=[
            pltpu.VMEM((b_per_w,), jnp.int32),
            pltpu.VMEM((b_per_w, d), jnp.float32),
            pltpu.SemaphoreType.DMA,
        ],
    )
    def k(table_hbm, idx_hbm, out_hbm, idx_v, rows_v, sem):
        wid = lax.axis_index("s") * info.num_cores + lax.axis_index("c")
        base = wid * b_per_w
        pltpu.sync_copy(idx_hbm.at[pl.ds(base, b_per_w)], idx_v)
        pltpu.async_copy(table_hbm.at[idx_v], rows_v, sem).wait()
        pltpu.sync_copy(rows_v, out_hbm.at[pl.ds(base, b_per_w)])

    return k(table, idx)


def _expert_body(d_ref, w1_ref, b1_ref, w2_ref, b2_ref, o_ref):
    f = pl.program_id(1)
    x = d_ref[0]
    h = (
        jnp.dot(x, w1_ref[0], preferred_element_type=jnp.float32) + b1_ref[0]
    )
    h = jax.nn.gelu(h).astype(jnp.bfloat16)
    part = jnp.dot(h, w2_ref[0], preferred_element_type=jnp.float32)

    @pl.when(f == 0)
    def _():
        o_ref[0] = part + b2_ref[0]

    @pl.when(f != 0)
    def _():
        o_ref[0] = o_ref[0] + part


def _expert_call(disp, w1, b1, w2, b2):
    return pl.pallas_call(
        _expert_body,
        grid=(E, FF // FBLK),
        in_specs=[
            pl.BlockSpec((1, CAP, H), lambda e, f: (e, 0, 0)),
            pl.BlockSpec((1, H, FBLK), lambda e, f: (e, 0, f)),
            pl.BlockSpec((1, 1, FBLK), lambda e, f: (e, 0, f)),
            pl.BlockSpec((1, FBLK, H), lambda e, f: (e, f, 0)),
            pl.BlockSpec((1, 1, H), lambda e, f: (e, 0, 0)),
        ],
        out_specs=pl.BlockSpec((1, CAP, H), lambda e, f: (e, 0, 0)),
        out_shape=jax.ShapeDtypeStruct((E, CAP, H), jnp.float32),
    )(disp, w1, b1, w2, b2)


def _combine_body(res_ref, rows_ref, g1_ref, o_ref):
    o_ref[...] = res_ref[...] + g1_ref[...] * rows_ref[...]


def _combine_call(res, rows, g1):
    return pl.pallas_call(
        _combine_body,
        grid=(S // TBLK,),
        in_specs=[
            pl.BlockSpec((TBLK, H), lambda i: (i, 0)),
            pl.BlockSpec((TBLK, H), lambda i: (i, 0)),
            pl.BlockSpec((TBLK, 1), lambda i: (i, 0)),
        ],
        out_specs=pl.BlockSpec((TBLK, H), lambda i: (i, 0)),
        out_shape=jax.ShapeDtypeStruct((S, H), jnp.float32),
    )(res, rows, g1)


def kernel(input, qkv_w, qkv_b, attn_ow, attn_ob, norm_w, norm_b,
           attn_nw, attn_nb, gate_w, inter_w, inter_b, output_w, output_b):
    x = input[0]

    qkv = _qkv_call(
        x, norm_w.reshape(1, H), norm_b.reshape(1, H), qkv_w,
        qkv_b.reshape(1, 3 * H),
    )
    q3 = qkv[:, 0:H].reshape(S, NH, DH).transpose(1, 0, 2)
    k3 = qkv[:, H:2 * H].reshape(S, NH, DH).transpose(1, 0, 2)
    v3 = qkv[:, 2 * H:3 * H].reshape(S, NH, DH).transpose(1, 0, 2)

    ctx = _attn_call(q3, k3, v3).transpose(1, 0, 2).reshape(S, H)

    gwp = jnp.pad(gate_w, ((0, 0), (0, 128 - E)))
    residual, hm, logits = _post_call(
        ctx, x, attn_ow, attn_ob.reshape(1, H), attn_nw.reshape(1, H),
        attn_nb.reshape(1, H), gwp,
    )

    g1, cidx, didx = _route_call(logits)

    disp = _sc_gather(hm, didx.reshape(S))
    eout = _expert_call(
        disp.reshape(E, CAP, H).astype(jnp.bfloat16),
        inter_w.astype(jnp.bfloat16), inter_b.reshape(E, 1, FF),
        output_w.astype(jnp.bfloat16), output_b.reshape(E, 1, H),
    )
    rows = _sc_gather(eout.reshape(E * CAP, H), cidx.reshape(S))

    out = _combine_call(residual, rows, g1)
    return out.reshape(1, S, H)


# speedup vs baseline: 1.2355x; 1.2355x over previous
import functools

import jax
import jax.numpy as jnp
from jax import lax
from jax.experimental import pallas as pl
from jax.experimental.pallas import tpu as pltpu
from jax.experimental.pallas import tpu_sc as plsc

S = 2048
H = 1024
NH = 16
DH = 64
E = 8
FF = 4096
CAP = 256
EPS = 1e-6
TBLK = 256
QBLK = 512
FBLK = 512


def _ln(x, w, b):
    mu = jnp.mean(x, axis=-1, keepdims=True)
    var = jnp.mean((x - mu) ** 2, axis=-1, keepdims=True)
    return (x - mu) / jnp.sqrt(var + EPS) * w + b


def _qkv_body(x_ref, nw_ref, nb_ref, w_ref, b_ref, o_ref):
    h = _ln(x_ref[...], nw_ref[...], nb_ref[...])
    o_ref[...] = (
        jnp.dot(h, w_ref[...], preferred_element_type=jnp.float32) + b_ref[...]
    )


def _qkv_call(x, nw, nb, w, b):
    return pl.pallas_call(
        _qkv_body,
        grid=(3, S // TBLK),
        in_specs=[
            pl.BlockSpec((TBLK, H), lambda j, i: (i, 0)),
            pl.BlockSpec((1, H), lambda j, i: (0, 0)),
            pl.BlockSpec((1, H), lambda j, i: (0, 0)),
            pl.BlockSpec((H, H), lambda j, i: (0, j)),
            pl.BlockSpec((1, H), lambda j, i: (0, j)),
        ],
        out_specs=pl.BlockSpec((TBLK, H), lambda j, i: (i, j)),
        out_shape=jax.ShapeDtypeStruct((S, 3 * H), jnp.float32),
    )(x, nw, nb, w, b)


def _attn_body(q_ref, k_ref, v_ref, o_ref):
    qb = pl.program_id(1)
    q = q_ref[0]
    q_idx = qb * QBLK + lax.broadcasted_iota(jnp.int32, (QBLK, QBLK), 0)
    k_iota = lax.broadcasted_iota(jnp.int32, (QBLK, QBLK), 1)

    def body(kb, carry):
        m, l, acc = carry
        k = k_ref[0, pl.ds(kb * QBLK, QBLK), :]
        v = v_ref[0, pl.ds(kb * QBLK, QBLK), :]
        s = lax.dot_general(
            q, k, (((1,), (1,)), ((), ())), preferred_element_type=jnp.float32
        ) * 0.125
        s = jnp.where(kb * QBLK + k_iota <= q_idx, s, jnp.float32(-1e9))
        m_new = jnp.maximum(m, jnp.max(s, axis=-1, keepdims=True))
        p = jnp.exp(s - m_new)
        alpha = jnp.exp(m - m_new)
        l_new = l * alpha + jnp.sum(p, axis=-1, keepdims=True)
        acc_new = acc * alpha + jnp.dot(
            p, v, preferred_element_type=jnp.float32
        )
        return m_new, l_new, acc_new

    init = (
        jnp.full((QBLK, 1), -1e30, jnp.float32),
        jnp.zeros((QBLK, 1), jnp.float32),
        jnp.zeros((QBLK, DH), jnp.float32),
    )
    m, l, acc = lax.fori_loop(0, qb + 1, body, init)
    o_ref[0] = acc / l


def _attn_call(q3, k3, v3):
    return pl.pallas_call(
        _attn_body,
        grid=(NH, S // QBLK),
        in_specs=[
            pl.BlockSpec((1, QBLK, DH), lambda h, i: (h, i, 0)),
            pl.BlockSpec((1, S, DH), lambda h, i: (h, 0, 0)),
            pl.BlockSpec((1, S, DH), lambda h, i: (h, 0, 0)),
        ],
        out_specs=pl.BlockSpec((1, QBLK, DH), lambda h, i: (h, i, 0)),
        out_shape=jax.ShapeDtypeStruct((NH, S, DH), jnp.float32),
    )(q3, k3, v3)


def _post_body(ctx_ref, x_ref, ow_ref, ob_ref, nw_ref, nb_ref, gw_ref,
               res_ref, hm_ref, lg_ref):
    attn_out = (
        jnp.dot(ctx_ref[...], ow_ref[...], preferred_element_type=jnp.float32)
        + ob_ref[...]
    )
    res = x_ref[...] + attn_out
    res_ref[...] = res
    hm = _ln(res, nw_ref[...], nb_ref[...])
    hm_ref[...] = hm
    lg_ref[...] = jnp.dot(hm, gw_ref[...], preferred_element_type=jnp.float32)


def _post_call(ctx, x, ow, ob, nw, nb, gwp):
    return pl.pallas_call(
        _post_body,
        grid=(S // TBLK,),
        in_specs=[
            pl.BlockSpec((TBLK, H), lambda i: (i, 0)),
            pl.BlockSpec((TBLK, H), lambda i: (i, 0)),
            pl.BlockSpec((H, H), lambda i: (0, 0)),
            pl.BlockSpec((1, H), lambda i: (0, 0)),
            pl.BlockSpec((1, H), lambda i: (0, 0)),
            pl.BlockSpec((1, H), lambda i: (0, 0)),
            pl.BlockSpec((H, 128), lambda i: (0, 0)),
        ],
        out_specs=[
            pl.BlockSpec((TBLK, H), lambda i: (i, 0)),
            pl.BlockSpec((TBLK, H), lambda i: (i, 0)),
            pl.BlockSpec((TBLK, 128), lambda i: (i, 0)),
        ],
        out_shape=[
            jax.ShapeDtypeStruct((S, H), jnp.float32),
            jax.ShapeDtypeStruct((S, H), jnp.float32),
            jax.ShapeDtypeStruct((S, 128), jnp.float32),
        ],
    )(ctx, x, ow, ob, nw, nb, gwp)


def _route_body(lg_ref, g1_ref, cidx_ref, didx_ref, slotk_ref):
    lane = lax.broadcasted_iota(jnp.int32, (TBLK, 128), 1)
    r = lax.broadcasted_iota(jnp.int32, (TBLK, TBLK), 0)
    c = lax.broadcasted_iota(jnp.int32, (TBLK, TBLK), 1)
    tri = (c < r).astype(jnp.float32)
    counts = jnp.zeros((1, 128), jnp.float32)
    for b in range(S // TBLK):
        sl = pl.ds(b * TBLK, TBLK)
        lg = jnp.where(lane < E, lg_ref[sl, :], jnp.float32(-1e30))
        m = jnp.max(lg, axis=1, keepdims=True)
        ex = jnp.exp(lg - m)
        g = ex / jnp.sum(ex, axis=1, keepdims=True)
        gm = jnp.max(g, axis=1, keepdims=True)
        is_max = jnp.logical_and(g == gm, lane < E)
        eidx = jnp.min(jnp.where(is_max, lane, 128), axis=1, keepdims=True)
        mask1 = (lane == eidx).astype(jnp.float32)
        loc_excl = (
            jnp.dot(tri, mask1, preferred_element_type=jnp.float32) + counts
        )
        counts = counts + jnp.sum(mask1, axis=0, keepdims=True)
        loc1 = jnp.sum(loc_excl * mask1, axis=1, keepdims=True)
        keep = loc1 < jnp.float32(CAP)
        g1 = jnp.sum(g * mask1, axis=1, keepdims=True)
        g1_ref[sl, :] = jnp.where(keep, g1, 0.0)
        slot = eidx * CAP + loc1.astype(jnp.int32)
        cidx_ref[sl, :] = jnp.where(keep, slot, 0)
        slotk_ref[sl, :] = jnp.where(keep, slot, -1)
    slotk = slotk_ref[...]
    for sb in range(S // TBLK):
        s_iota = sb * TBLK + lax.broadcasted_iota(jnp.int32, (S, TBLK), 1)
        eq = slotk == s_iota
        t_col = lax.broadcasted_iota(jnp.int32, (S, TBLK), 0)
        inv = jnp.sum(jnp.where(eq, t_col, 0), axis=0, keepdims=True)
        didx_ref[:, pl.ds(sb * TBLK, TBLK)] = inv


def _route_call(logits):
    return pl.pallas_call(
        _route_body,
        grid=(1,),
        in_specs=[pl.BlockSpec((S, 128), lambda i: (0, 0))],
        out_specs=[
            pl.BlockSpec((S, 1), lambda i: (0, 0)),
            pl.BlockSpec((S, 1), lambda i: (0, 0)),
            pl.BlockSpec((1, S), lambda i: (0, 0)),
        ],
        out_shape=[
            jax.ShapeDtypeStruct((S, 1), jnp.float32),
            jax.ShapeDtypeStruct((S, 1), jnp.int32),
            jax.ShapeDtypeStruct((1, S), jnp.int32),
        ],
        scratch_shapes=[pltpu.VMEM((S, 1), jnp.int32)],
    )(logits)


def _sc_gather(table, idx):
    info = plsc.get_sparse_core_info()
    nw = info.num_cores * info.num_subcores
    b = idx.shape[0]
    d = table.shape[1]
    b_per_w = b // nw
    mesh = plsc.VectorSubcoreMesh(core_axis_name="c", subcore_axis_name="s")

    @functools.partial(
        pl.kernel,
        mesh=mesh,
        out_type=jax.ShapeDtypeStruct((b, d), jnp.float32),
        scratch_types=[
            pltpu.VMEM((b_per_w,), jnp.int32),
            pltpu.VMEM((b_per_w, d), jnp.float32),
            pltpu.SemaphoreType.DMA,
        ],
    )
    def k(table_hbm, idx_hbm, out_hbm, idx_v, rows_v, sem):
        wid = lax.axis_index("s") * info.num_cores + lax.axis_index("c")
        base = wid * b_per_w
        pltpu.sync_copy(idx_hbm.at[pl.ds(base, b_per_w)], idx_v)
        pltpu.async_copy(table_hbm.at[idx_v], rows_v, sem).wait()
        pltpu.sync_copy(rows_v, out_hbm.at[pl.ds(base, b_per_w)])

    return k(table, idx)


def _expert_body(d_ref, w1_ref, b1_ref, w2_ref, b2_ref, o_ref):
    f = pl.program_id(1)
    x = d_ref[0].astype(jnp.bfloat16)
    w1 = w1_ref[0].astype(jnp.bfloat16)
    h = jnp.dot(x, w1, preferred_element_type=jnp.float32) + b1_ref[0]
    h = jax.nn.gelu(h).astype(jnp.bfloat16)
    w2 = w2_ref[0].astype(jnp.bfloat16)
    part = jnp.dot(h, w2, preferred_element_type=jnp.float32)

    @pl.when(f == 0)
    def _():
        o_ref[0] = part + b2_ref[0]

    @pl.when(f != 0)
    def _():
        o_ref[0] = o_ref[0] + part


def _expert_call(disp, w1, b1, w2, b2):
    return pl.pallas_call(
        _expert_body,
        grid=(E, FF // FBLK),
        in_specs=[
            pl.BlockSpec((1, CAP, H), lambda e, f: (e, 0, 0)),
            pl.BlockSpec((1, H, FBLK), lambda e, f: (e, 0, f)),
            pl.BlockSpec((1, 1, FBLK), lambda e, f: (e, 0, f)),
            pl.BlockSpec((1, FBLK, H), lambda e, f: (e, f, 0)),
            pl.BlockSpec((1, 1, H), lambda e, f: (e, 0, 0)),
        ],
        out_specs=pl.BlockSpec((1, CAP, H), lambda e, f: (e, 0, 0)),
        out_shape=jax.ShapeDtypeStruct((E, CAP, H), jnp.float32),
    )(disp, w1, b1, w2, b2)


def _combine_body(res_ref, rows_ref, g1_ref, o_ref):
    o_ref[...] = res_ref[...] + g1_ref[...] * rows_ref[...]


def _combine_call(res, rows, g1):
    return pl.pallas_call(
        _combine_body,
        grid=(S // TBLK,),
        in_specs=[
            pl.BlockSpec((TBLK, H), lambda i: (i, 0)),
            pl.BlockSpec((TBLK, H), lambda i: (i, 0)),
            pl.BlockSpec((TBLK, 1), lambda i: (i, 0)),
        ],
        out_specs=pl.BlockSpec((TBLK, H), lambda i: (i, 0)),
        out_shape=jax.ShapeDtypeStruct((S, H), jnp.float32),
    )(res, rows, g1)


def kernel(input, qkv_w, qkv_b, attn_ow, attn_ob, norm_w, norm_b,
           attn_nw, attn_nb, gate_w, inter_w, inter_b, output_w, output_b):
    x = input[0]

    qkv = _qkv_call(
        x, norm_w.reshape(1, H), norm_b.reshape(1, H), qkv_w,
        qkv_b.reshape(1, 3 * H),
    )
    q3 = qkv[:, 0:H].reshape(S, NH, DH).transpose(1, 0, 2)
    k3 = qkv[:, H:2 * H].reshape(S, NH, DH).transpose(1, 0, 2)
    v3 = qkv[:, 2 * H:3 * H].reshape(S, NH, DH).transpose(1, 0, 2)

    ctx = _attn_call(q3, k3, v3).transpose(1, 0, 2).reshape(S, H)

    gwp = jnp.pad(gate_w, ((0, 0), (0, 128 - E)))
    residual, hm, logits = _post_call(
        ctx, x, attn_ow, attn_ob.reshape(1, H), attn_nw.reshape(1, H),
        attn_nb.reshape(1, H), gwp,
    )

    g1, cidx, didx = _route_call(logits)

    disp = _sc_gather(hm, didx.reshape(S))
    eout = _expert_call(
        disp.reshape(E, CAP, H), inter_w, inter_b.reshape(E, 1, FF),
        output_w, output_b.reshape(E, 1, H),
    )
    rows = _sc_gather(eout.reshape(E * CAP, H), cidx.reshape(S))

    out = _combine_call(residual, rows, g1)
    return out.reshape(1, S, H)


# speedup vs baseline: 1.2370x; 1.0013x over previous
import functools

import jax
import jax.numpy as jnp
from jax import lax
from jax.experimental import pallas as pl
from jax.experimental.pallas import tpu as pltpu
from jax.experimental.pallas import tpu_sc as plsc

S = 2048
H = 1024
NH = 16
DH = 64
E = 8
FF = 4096
CAP = 256
EPS = 1e-6
TBLK = 256
QBLK = 512
FBLK = 512


def _ln(x, w, b):
    mu = jnp.mean(x, axis=-1, keepdims=True)
    var = jnp.mean((x - mu) ** 2, axis=-1, keepdims=True)
    return (x - mu) / jnp.sqrt(var + EPS) * w + b


def _qkv_body(x_ref, nw_ref, nb_ref, w_ref, b_ref, o_ref):
    h = _ln(x_ref[...], nw_ref[...], nb_ref[...])
    o_ref[...] = (
        jnp.dot(h, w_ref[...], preferred_element_type=jnp.float32) + b_ref[...]
    )


def _qkv_call(x, nw, nb, w, b):
    return pl.pallas_call(
        _qkv_body,
        grid=(3, S // TBLK),
        in_specs=[
            pl.BlockSpec((TBLK, H), lambda j, i: (i, 0)),
            pl.BlockSpec((1, H), lambda j, i: (0, 0)),
            pl.BlockSpec((1, H), lambda j, i: (0, 0)),
            pl.BlockSpec((H, H), lambda j, i: (0, j)),
            pl.BlockSpec((1, H), lambda j, i: (0, j)),
        ],
        out_specs=pl.BlockSpec((TBLK, H), lambda j, i: (i, j)),
        out_shape=jax.ShapeDtypeStruct((S, 3 * H), jnp.float32),
    )(x, nw, nb, w, b)


def _attn_body(q_ref, k_ref, v_ref, o_ref):
    qb = pl.program_id(1)
    q = q_ref[0]
    q_idx = qb * QBLK + lax.broadcasted_iota(jnp.int32, (QBLK, QBLK), 0)
    k_iota = lax.broadcasted_iota(jnp.int32, (QBLK, QBLK), 1)

    def body(kb, carry):
        m, l, acc = carry
        k = k_ref[0, pl.ds(kb * QBLK, QBLK), :]
        v = v_ref[0, pl.ds(kb * QBLK, QBLK), :]
        s = lax.dot_general(
            q, k, (((1,), (1,)), ((), ())), preferred_element_type=jnp.float32
        ) * 0.125
        s = jnp.where(kb * QBLK + k_iota <= q_idx, s, jnp.float32(-1e9))
        m_new = jnp.maximum(m, jnp.max(s, axis=-1, keepdims=True))
        p = jnp.exp(s - m_new)
        alpha = jnp.exp(m - m_new)
        l_new = l * alpha + jnp.sum(p, axis=-1, keepdims=True)
        acc_new = acc * alpha + jnp.dot(
            p, v, preferred_element_type=jnp.float32
        )
        return m_new, l_new, acc_new

    init = (
        jnp.full((QBLK, 1), -1e30, jnp.float32),
        jnp.zeros((QBLK, 1), jnp.float32),
        jnp.zeros((QBLK, DH), jnp.float32),
    )
    m, l, acc = lax.fori_loop(0, qb + 1, body, init)
    o_ref[0] = acc / l


def _attn_call(q3, k3, v3):
    return pl.pallas_call(
        _attn_body,
        grid=(NH, S // QBLK),
        in_specs=[
            pl.BlockSpec((1, QBLK, DH), lambda h, i: (h, i, 0)),
            pl.BlockSpec((1, S, DH), lambda h, i: (h, 0, 0)),
            pl.BlockSpec((1, S, DH), lambda h, i: (h, 0, 0)),
        ],
        out_specs=pl.BlockSpec((1, QBLK, DH), lambda h, i: (h, i, 0)),
        out_shape=jax.ShapeDtypeStruct((NH, S, DH), jnp.float32),
    )(q3, k3, v3)


def _post_body(ctx_ref, x_ref, ow_ref, ob_ref, nw_ref, nb_ref, gw_ref,
               res_ref, hm_ref, lg_ref):
    attn_out = (
        jnp.dot(ctx_ref[...], ow_ref[...], preferred_element_type=jnp.float32)
        + ob_ref[...]
    )
    res = x_ref[...] + attn_out
    res_ref[...] = res
    hm = _ln(res, nw_ref[...], nb_ref[...])
    hm_ref[...] = hm
    lg_ref[...] = jnp.dot(hm, gw_ref[...], preferred_element_type=jnp.float32)


def _post_call(ctx, x, ow, ob, nw, nb, gwp):
    return pl.pallas_call(
        _post_body,
        grid=(S // TBLK,),
        in_specs=[
            pl.BlockSpec((TBLK, H), lambda i: (i, 0)),
            pl.BlockSpec((TBLK, H), lambda i: (i, 0)),
            pl.BlockSpec((H, H), lambda i: (0, 0)),
            pl.BlockSpec((1, H), lambda i: (0, 0)),
            pl.BlockSpec((1, H), lambda i: (0, 0)),
            pl.BlockSpec((1, H), lambda i: (0, 0)),
            pl.BlockSpec((H, 128), lambda i: (0, 0)),
        ],
        out_specs=[
            pl.BlockSpec((TBLK, H), lambda i: (i, 0)),
            pl.BlockSpec((TBLK, H), lambda i: (i, 0)),
            pl.BlockSpec((TBLK, 128), lambda i: (i, 0)),
        ],
        out_shape=[
            jax.ShapeDtypeStruct((S, H), jnp.float32),
            jax.ShapeDtypeStruct((S, H), jnp.float32),
            jax.ShapeDtypeStruct((S, 128), jnp.float32),
        ],
    )(ctx, x, ow, ob, nw, nb, gwp)


def _route_body(lg_ref, g1_ref, cidx_ref, didx_ref, slotk_ref):
    lane = lax.broadcasted_iota(jnp.int32, (TBLK, 128), 1)
    r = lax.broadcasted_iota(jnp.int32, (TBLK, TBLK), 0)
    c = lax.broadcasted_iota(jnp.int32, (TBLK, TBLK), 1)
    tri = (c < r).astype(jnp.float32)
    counts = jnp.zeros((1, 128), jnp.float32)
    for b in range(S // TBLK):
        sl = pl.ds(b * TBLK, TBLK)
        lg = jnp.where(lane < E, lg_ref[sl, :], jnp.float32(-1e30))
        m = jnp.max(lg, axis=1, keepdims=True)
        ex = jnp.exp(lg - m)
        g = ex / jnp.sum(ex, axis=1, keepdims=True)
        gm = jnp.max(g, axis=1, keepdims=True)
        is_max = jnp.logical_and(g == gm, lane < E)
        eidx = jnp.min(jnp.where(is_max, lane, 128), axis=1, keepdims=True)
        mask1 = (lane == eidx).astype(jnp.float32)
        loc_excl = (
            jnp.dot(tri, mask1, preferred_element_type=jnp.float32) + counts
        )
        counts = counts + jnp.sum(mask1, axis=0, keepdims=True)
        loc1 = jnp.sum(loc_excl * mask1, axis=1, keepdims=True)
        keep = loc1 < jnp.float32(CAP)
        g1 = jnp.sum(g * mask1, axis=1, keepdims=True)
        g1_ref[sl, :] = jnp.where(keep, g1, 0.0)
        slot = eidx * CAP + loc1.astype(jnp.int32)
        cidx_ref[sl, :] = jnp.where(keep, slot, 0)
        slotk_ref[sl, :] = jnp.where(keep, slot, -1)
    slotk = slotk_ref[...]
    for sb in range(S // TBLK):
        s_iota = sb * TBLK + lax.broadcasted_iota(jnp.int32, (S, TBLK), 1)
        eq = slotk == s_iota
        t_col = lax.broadcasted_iota(jnp.int32, (S, TBLK), 0)
        inv = jnp.sum(jnp.where(eq, t_col, 0), axis=0, keepdims=True)
        didx_ref[:, pl.ds(sb * TBLK, TBLK)] = inv


def _route_call(logits):
    return pl.pallas_call(
        _route_body,
        grid=(1,),
        in_specs=[pl.BlockSpec((S, 128), lambda i: (0, 0))],
        out_specs=[
            pl.BlockSpec((S, 1), lambda i: (0, 0)),
            pl.BlockSpec((S, 1), lambda i: (0, 0)),
            pl.BlockSpec((1, S), lambda i: (0, 0)),
        ],
        out_shape=[
            jax.ShapeDtypeStruct((S, 1), jnp.float32),
            jax.ShapeDtypeStruct((S, 1), jnp.int32),
            jax.ShapeDtypeStruct((1, S), jnp.int32),
        ],
        scratch_shapes=[pltpu.VMEM((S, 1), jnp.int32)],
    )(logits)


def _sc_gather(table, idx):
    info = plsc.get_sparse_core_info()
    nw = info.num_cores * info.num_subcores
    b = idx.shape[0]
    d = table.shape[1]
    b_per_w = b // nw
    mesh = plsc.VectorSubcoreMesh(core_axis_name="c", subcore_axis_name="s")

    @functools.partial(
        pl.kernel,
        mesh=mesh,
        out_type=jax.ShapeDtypeStruct((b, d), jnp.float32),
        scratch_types=[
            pltpu.VMEM((b_per_w,), jnp.int32),
            pltpu.VMEM((b_per_w, d), jnp.float32),
            pltpu.SemaphoreType.DMA,
        ],
    )
    def k(table_hbm, idx_hbm, out_hbm, idx_v, rows_v, sem):
        wid = lax.axis_index("s") * info.num_cores + lax.axis_index("c")
        base = wid * b_per_w
        pltpu.sync_copy(idx_hbm.at[pl.ds(base, b_per_w)], idx_v)
        pltpu.async_copy(table_hbm.at[idx_v], rows_v, sem).wait()
        pltpu.sync_copy(rows_v, out_hbm.at[pl.ds(base, b_per_w)])

    return k(table, idx)


def _expert_body(d_ref, w1_ref, b1_ref, w2_ref, b2_ref, o_ref):
    f = pl.program_id(1)
    x = d_ref[0]
    h = (
        jnp.dot(x, w1_ref[0], preferred_element_type=jnp.float32) + b1_ref[0]
    )
    h = jax.nn.gelu(h)
    part = jnp.dot(h, w2_ref[0], preferred_element_type=jnp.float32)

    @pl.when(f == 0)
    def _():
        o_ref[0] = part + b2_ref[0]

    @pl.when(f != 0)
    def _():
        o_ref[0] = o_ref[0] + part


def _expert_call(disp, w1, b1, w2, b2):
    return pl.pallas_call(
        _expert_body,
        grid=(E, FF // FBLK),
        in_specs=[
            pl.BlockSpec((1, CAP, H), lambda e, f: (e, 0, 0)),
            pl.BlockSpec((1, H, FBLK), lambda e, f: (e, 0, f)),
            pl.BlockSpec((1, 1, FBLK), lambda e, f: (e, 0, f)),
            pl.BlockSpec((1, FBLK, H), lambda e, f: (e, f, 0)),
            pl.BlockSpec((1, 1, H), lambda e, f: (e, 0, 0)),
        ],
        out_specs=pl.BlockSpec((1, CAP, H), lambda e, f: (e, 0, 0)),
        out_shape=jax.ShapeDtypeStruct((E, CAP, H), jnp.float32),
    )(disp, w1, b1, w2, b2)


def _combine_body(res_ref, rows_ref, g1_ref, o_ref):
    o_ref[...] = res_ref[...] + g1_ref[...] * rows_ref[...]


def _combine_call(res, rows, g1):
    return pl.pallas_call(
        _combine_body,
        grid=(S // TBLK,),
        in_specs=[
            pl.BlockSpec((TBLK, H), lambda i: (i, 0)),
            pl.BlockSpec((TBLK, H), lambda i: (i, 0)),
            pl.BlockSpec((TBLK, 1), lambda i: (i, 0)),
        ],
        out_specs=pl.BlockSpec((TBLK, H), lambda i: (i, 0)),
        out_shape=jax.ShapeDtypeStruct((S, H), jnp.float32),
    )(res, rows, g1)


def kernel(input, qkv_w, qkv_b, attn_ow, attn_ob, norm_w, norm_b,
           attn_nw, attn_nb, gate_w, inter_w, inter_b, output_w, output_b):
    x = input[0]

    qkv = _qkv_call(
        x, norm_w.reshape(1, H), norm_b.reshape(1, H), qkv_w,
        qkv_b.reshape(1, 3 * H),
    )
    q3 = qkv[:, 0:H].reshape(S, NH, DH).transpose(1, 0, 2)
    k3 = qkv[:, H:2 * H].reshape(S, NH, DH).transpose(1, 0, 2)
    v3 = qkv[:, 2 * H:3 * H].reshape(S, NH, DH).transpose(1, 0, 2)

    ctx = _attn_call(q3, k3, v3).transpose(1, 0, 2).reshape(S, H)

    gwp = jnp.pad(gate_w, ((0, 0), (0, 128 - E)))
    residual, hm, logits = _post_call(
        ctx, x, attn_ow, attn_ob.reshape(1, H), attn_nw.reshape(1, H),
        attn_nb.reshape(1, H), gwp,
    )

    g1, cidx, didx = _route_call(logits)

    disp = _sc_gather(hm, didx.reshape(S))
    eout = _expert_call(
        disp.reshape(E, CAP, H), inter_w, inter_b.reshape(E, 1, FF),
        output_w, output_b.reshape(E, 1, H),
    )
    rows = _sc_gather(eout.reshape(E * CAP, H), cidx.reshape(S))

    out = _combine_call(residual, rows, g1)
    return out.reshape(1, S, H)


# speedup vs baseline: 1.5057x; 1.2172x over previous
import functools

import jax
import jax.numpy as jnp
from jax import lax
from jax.experimental import pallas as pl
from jax.experimental.pallas import tpu as pltpu
from jax.experimental.pallas import tpu_sc as plsc

S = 2048
H = 1024
NH = 16
DH = 64
E = 8
FF = 4096
CAP = 256
EPS = 1e-6
TBLK = 256
QBLK = 512
FBLK = 512


def _ln(x, w, b):
    mu = jnp.mean(x, axis=-1, keepdims=True)
    var = jnp.mean((x - mu) ** 2, axis=-1, keepdims=True)
    return (x - mu) / jnp.sqrt(var + EPS) * w + b


def _qkv_body(x_ref, nw_ref, nb_ref, w_ref, b_ref, o_ref):
    h = _ln(x_ref[...], nw_ref[...], nb_ref[...])
    o_ref[...] = (
        jnp.dot(h, w_ref[...], preferred_element_type=jnp.float32) + b_ref[...]
    )


def _qkv_call(x, nw, nb, w, b):
    return pl.pallas_call(
        _qkv_body,
        grid=(3, S // TBLK),
        in_specs=[
            pl.BlockSpec((TBLK, H), lambda j, i: (i, 0)),
            pl.BlockSpec((1, H), lambda j, i: (0, 0)),
            pl.BlockSpec((1, H), lambda j, i: (0, 0)),
            pl.BlockSpec((H, H), lambda j, i: (0, j)),
            pl.BlockSpec((1, H), lambda j, i: (0, j)),
        ],
        out_specs=pl.BlockSpec((TBLK, H), lambda j, i: (i, j)),
        out_shape=jax.ShapeDtypeStruct((S, 3 * H), jnp.float32),
    )(x, nw, nb, w, b)


def _attn_body(q_ref, k_ref, v_ref, o_ref):
    qb = pl.program_id(1)
    q_idx = qb * QBLK + lax.broadcasted_iota(jnp.int32, (QBLK, QBLK), 0)
    k_iota = lax.broadcasted_iota(jnp.int32, (QBLK, QBLK), 1)
    outs = []
    for hh in range(2):
        q = q_ref[:, pl.ds(hh * DH, DH)]

        def body(kb, carry):
            m, l, acc = carry
            k = k_ref[pl.ds(kb * QBLK, QBLK), pl.ds(hh * DH, DH)]
            v = v_ref[pl.ds(kb * QBLK, QBLK), pl.ds(hh * DH, DH)]
            s = lax.dot_general(
                q, k, (((1,), (1,)), ((), ())),
                preferred_element_type=jnp.float32,
            ) * 0.125
            s = jnp.where(kb * QBLK + k_iota <= q_idx, s, jnp.float32(-1e9))
            m_new = jnp.maximum(m, jnp.max(s, axis=-1, keepdims=True))
            p = jnp.exp(s - m_new)
            alpha = jnp.exp(m - m_new)
            l_new = l * alpha + jnp.sum(p, axis=-1, keepdims=True)
            acc_new = acc * alpha + jnp.dot(
                p, v, preferred_element_type=jnp.float32
            )
            return m_new, l_new, acc_new

        init = (
            jnp.full((QBLK, 1), -1e30, jnp.float32),
            jnp.zeros((QBLK, 1), jnp.float32),
            jnp.zeros((QBLK, DH), jnp.float32),
        )
        m, l, acc = lax.fori_loop(0, qb + 1, body, init)
        outs.append(acc / l)
    o_ref[...] = jnp.concatenate(outs, axis=1)


def _attn_call(qkv):
    return pl.pallas_call(
        _attn_body,
        grid=(NH // 2, S // QBLK),
        in_specs=[
            pl.BlockSpec((QBLK, 2 * DH), lambda h, i: (i, h)),
            pl.BlockSpec((S, 2 * DH), lambda h, i: (0, NH // 2 + h)),
            pl.BlockSpec((S, 2 * DH), lambda h, i: (0, NH + h)),
        ],
        out_specs=pl.BlockSpec((QBLK, 2 * DH), lambda h, i: (i, h)),
        out_shape=jax.ShapeDtypeStruct((S, H), jnp.float32),
    )(qkv, qkv, qkv)


def _post_body(ctx_ref, x_ref, ow_ref, ob_ref, nw_ref, nb_ref, gw_ref,
               res_ref, hm_ref, lg_ref):
    attn_out = (
        jnp.dot(ctx_ref[...], ow_ref[...], preferred_element_type=jnp.float32)
        + ob_ref[...]
    )
    res = x_ref[...] + attn_out
    res_ref[...] = res
    hm = _ln(res, nw_ref[...], nb_ref[...])
    hm_ref[...] = hm
    lg_ref[...] = jnp.dot(hm, gw_ref[...], preferred_element_type=jnp.float32)


def _post_call(ctx, x, ow, ob, nw, nb, gwp):
    return pl.pallas_call(
        _post_body,
        grid=(S // TBLK,),
        in_specs=[
            pl.BlockSpec((TBLK, H), lambda i: (i, 0)),
            pl.BlockSpec((TBLK, H), lambda i: (i, 0)),
            pl.BlockSpec((H, H), lambda i: (0, 0)),
            pl.BlockSpec((1, H), lambda i: (0, 0)),
            pl.BlockSpec((1, H), lambda i: (0, 0)),
            pl.BlockSpec((1, H), lambda i: (0, 0)),
            pl.BlockSpec((H, 128), lambda i: (0, 0)),
        ],
        out_specs=[
            pl.BlockSpec((TBLK, H), lambda i: (i, 0)),
            pl.BlockSpec((TBLK, H), lambda i: (i, 0)),
            pl.BlockSpec((TBLK, 128), lambda i: (i, 0)),
        ],
        out_shape=[
            jax.ShapeDtypeStruct((S, H), jnp.float32),
            jax.ShapeDtypeStruct((S, H), jnp.float32),
            jax.ShapeDtypeStruct((S, 128), jnp.float32),
        ],
    )(ctx, x, ow, ob, nw, nb, gwp)


def _route_body(lg_ref, g1_ref, cidx_ref, didx_ref, slotk_ref):
    lane = lax.broadcasted_iota(jnp.int32, (TBLK, 128), 1)
    r = lax.broadcasted_iota(jnp.int32, (TBLK, TBLK), 0)
    c = lax.broadcasted_iota(jnp.int32, (TBLK, TBLK), 1)
    tri = (c < r).astype(jnp.float32)
    counts = jnp.zeros((1, 128), jnp.float32)
    for b in range(S // TBLK):
        sl = pl.ds(b * TBLK, TBLK)
        lg = jnp.where(lane < E, lg_ref[sl, :], jnp.float32(-1e30))
        m = jnp.max(lg, axis=1, keepdims=True)
        ex = jnp.exp(lg - m)
        g = ex / jnp.sum(ex, axis=1, keepdims=True)
        gm = jnp.max(g, axis=1, keepdims=True)
        is_max = jnp.logical_and(g == gm, lane < E)
        eidx = jnp.min(jnp.where(is_max, lane, 128), axis=1, keepdims=True)
        mask1 = (lane == eidx).astype(jnp.float32)
        loc_excl = (
            jnp.dot(tri, mask1, preferred_element_type=jnp.float32) + counts
        )
        counts = counts + jnp.sum(mask1, axis=0, keepdims=True)
        loc1 = jnp.sum(loc_excl * mask1, axis=1, keepdims=True)
        keep = loc1 < jnp.float32(CAP)
        g1 = jnp.sum(g * mask1, axis=1, keepdims=True)
        g1_ref[sl, :] = jnp.where(keep, g1, 0.0)
        slot = eidx * CAP + loc1.astype(jnp.int32)
        cidx_ref[sl, :] = jnp.where(keep, slot, 0)
        slotk_ref[sl, :] = jnp.where(keep, slot, -1)
    slotk = slotk_ref[...]
    for sb in range(S // TBLK):
        s_iota = sb * TBLK + lax.broadcasted_iota(jnp.int32, (S, TBLK), 1)
        eq = slotk == s_iota
        t_col = lax.broadcasted_iota(jnp.int32, (S, TBLK), 0)
        inv = jnp.sum(jnp.where(eq, t_col, 0), axis=0, keepdims=True)
        didx_ref[:, pl.ds(sb * TBLK, TBLK)] = inv


def _route_call(logits):
    return pl.pallas_call(
        _route_body,
        grid=(1,),
        in_specs=[pl.BlockSpec((S, 128), lambda i: (0, 0))],
        out_specs=[
            pl.BlockSpec((S, 1), lambda i: (0, 0)),
            pl.BlockSpec((S, 1), lambda i: (0, 0)),
            pl.BlockSpec((1, S), lambda i: (0, 0)),
        ],
        out_shape=[
            jax.ShapeDtypeStruct((S, 1), jnp.float32),
            jax.ShapeDtypeStruct((S, 1), jnp.int32),
            jax.ShapeDtypeStruct((1, S), jnp.int32),
        ],
        scratch_shapes=[pltpu.VMEM((S, 1), jnp.int32)],
    )(logits)


def _sc_gather(table, idx):
    info = plsc.get_sparse_core_info()
    nw = info.num_cores * info.num_subcores
    b = idx.shape[0]
    d = table.shape[1]
    b_per_w = b // nw
    mesh = plsc.VectorSubcoreMesh(core_axis_name="c", subcore_axis_name="s")

    @functools.partial(
        pl.kernel,
        mesh=mesh,
        out_type=jax.ShapeDtypeStruct((b, d), jnp.float32),
        scratch_types=[
            pltpu.VMEM((b_per_w,), jnp.int32),
            pltpu.VMEM((b_per_w, d), jnp.float32),
            pltpu.SemaphoreType.DMA,
        ],
    )
    def k(table_hbm, idx_hbm, out_hbm, idx_v, rows_v, sem):
        wid = lax.axis_index("s") * info.num_cores + lax.axis_index("c")
        base = wid * b_per_w
        pltpu.sync_copy(idx_hbm.at[pl.ds(base, b_per_w)], idx_v)
        pltpu.async_copy(table_hbm.at[idx_v], rows_v, sem).wait()
        pltpu.sync_copy(rows_v, out_hbm.at[pl.ds(base, b_per_w)])

    return k(table, idx)


def _expert_body(d_ref, w1_ref, b1_ref, w2_ref, b2_ref, o_ref):
    f = pl.program_id(1)
    x = d_ref[0]
    h = (
        jnp.dot(x, w1_ref[0], preferred_element_type=jnp.float32) + b1_ref[0]
    )
    h = jax.nn.gelu(h)
    part = jnp.dot(h, w2_ref[0], preferred_element_type=jnp.float32)

    @pl.when(f == 0)
    def _():
        o_ref[0] = part + b2_ref[0]

    @pl.when(f != 0)
    def _():
        o_ref[0] = o_ref[0] + part


def _expert_call(disp, w1, b1, w2, b2):
    return pl.pallas_call(
        _expert_body,
        grid=(E, FF // FBLK),
        in_specs=[
            pl.BlockSpec((1, CAP, H), lambda e, f: (e, 0, 0)),
            pl.BlockSpec((1, H, FBLK), lambda e, f: (e, 0, f)),
            pl.BlockSpec((1, 1, FBLK), lambda e, f: (e, 0, f)),
            pl.BlockSpec((1, FBLK, H), lambda e, f: (e, f, 0)),
            pl.BlockSpec((1, 1, H), lambda e, f: (e, 0, 0)),
        ],
        out_specs=pl.BlockSpec((1, CAP, H), lambda e, f: (e, 0, 0)),
        out_shape=jax.ShapeDtypeStruct((E, CAP, H), jnp.float32),
    )(disp, w1, b1, w2, b2)


def _combine_body(res_ref, rows_ref, g1_ref, o_ref):
    o_ref[...] = res_ref[...] + g1_ref[...] * rows_ref[...]


def _combine_call(res, rows, g1):
    return pl.pallas_call(
        _combine_body,
        grid=(S // TBLK,),
        in_specs=[
            pl.BlockSpec((TBLK, H), lambda i: (i, 0)),
            pl.BlockSpec((TBLK, H), lambda i: (i, 0)),
            pl.BlockSpec((TBLK, 1), lambda i: (i, 0)),
        ],
        out_specs=pl.BlockSpec((TBLK, H), lambda i: (i, 0)),
        out_shape=jax.ShapeDtypeStruct((S, H), jnp.float32),
    )(res, rows, g1)


def kernel(input, qkv_w, qkv_b, attn_ow, attn_ob, norm_w, norm_b,
           attn_nw, attn_nb, gate_w, inter_w, inter_b, output_w, output_b):
    x = input[0]

    qkv = _qkv_call(
        x, norm_w.reshape(1, H), norm_b.reshape(1, H), qkv_w,
        qkv_b.reshape(1, 3 * H),
    )
    ctx = _attn_call(qkv)

    gwp = jnp.pad(gate_w, ((0, 0), (0, 128 - E)))
    residual, hm, logits = _post_call(
        ctx, x, attn_ow, attn_ob.reshape(1, H), attn_nw.reshape(1, H),
        attn_nb.reshape(1, H), gwp,
    )

    g1, cidx, didx = _route_call(logits)

    disp = _sc_gather(hm, didx.reshape(S))
    eout = _expert_call(
        disp.reshape(E, CAP, H), inter_w, inter_b.reshape(E, 1, FF),
        output_w, output_b.reshape(E, 1, H),
    )
    rows = _sc_gather(eout.reshape(E * CAP, H), cidx.reshape(S))

    out = _combine_call(residual, rows, g1)
    return out.reshape(1, S, H)


# speedup vs baseline: 1.6548x; 1.0990x over previous
import functools

import jax
import jax.numpy as jnp
from jax import lax
from jax.experimental import pallas as pl
from jax.experimental.pallas import tpu as pltpu
from jax.experimental.pallas import tpu_sc as plsc

S = 2048
H = 1024
NH = 16
DH = 64
E = 8
FF = 4096
CAP = 256
EPS = 1e-6
TBLK = 256
QBLK = 512
FBLK = 512


def _ln(x, w, b):
    mu = jnp.mean(x, axis=-1, keepdims=True)
    var = jnp.mean((x - mu) ** 2, axis=-1, keepdims=True)
    return (x - mu) / jnp.sqrt(var + EPS) * w + b


def _qkv_body(x_ref, nw_ref, nb_ref, w_ref, b_ref, o_ref):
    h = _ln(x_ref[...], nw_ref[...], nb_ref[...])
    o_ref[...] = (
        jnp.dot(h, w_ref[...], preferred_element_type=jnp.float32) + b_ref[...]
    )


def _qkv_call(x, nw, nb, w, b):
    return pl.pallas_call(
        _qkv_body,
        grid=(3, S // TBLK),
        in_specs=[
            pl.BlockSpec((TBLK, H), lambda j, i: (i, 0)),
            pl.BlockSpec((1, H), lambda j, i: (0, 0)),
            pl.BlockSpec((1, H), lambda j, i: (0, 0)),
            pl.BlockSpec((H, H), lambda j, i: (0, j)),
            pl.BlockSpec((1, H), lambda j, i: (0, j)),
        ],
        out_specs=pl.BlockSpec((TBLK, H), lambda j, i: (i, j)),
        out_shape=jax.ShapeDtypeStruct((S, 3 * H), jnp.float32),
    )(x, nw, nb, w, b)


def _attn_body(q_ref, k_ref, v_ref, o_ref):
    qb = pl.program_id(1)
    q_idx = qb * QBLK + lax.broadcasted_iota(jnp.int32, (QBLK, QBLK), 0)
    k_iota = lax.broadcasted_iota(jnp.int32, (QBLK, QBLK), 1)
    outs = []
    for hh in range(2):
        q = q_ref[:, pl.ds(hh * DH, DH)]

        def body(kb, carry):
            l, acc = carry
            k = k_ref[pl.ds(kb * QBLK, QBLK), pl.ds(hh * DH, DH)]
            v = v_ref[pl.ds(kb * QBLK, QBLK), pl.ds(hh * DH, DH)]
            s = lax.dot_general(
                q, k, (((1,), (1,)), ((), ())),
                preferred_element_type=jnp.float32,
            ) * 0.125
            e = jnp.exp(s)
            l_new = l + jnp.sum(e, axis=-1, keepdims=True)
            acc_new = acc + jnp.dot(
                e, v, preferred_element_type=jnp.float32
            )
            return l_new, acc_new

        init = (
            jnp.zeros((QBLK, 1), jnp.float32),
            jnp.zeros((QBLK, DH), jnp.float32),
        )
        l, acc = lax.fori_loop(0, qb, body, init)
        k = k_ref[pl.ds(qb * QBLK, QBLK), pl.ds(hh * DH, DH)]
        v = v_ref[pl.ds(qb * QBLK, QBLK), pl.ds(hh * DH, DH)]
        s = lax.dot_general(
            q, k, (((1,), (1,)), ((), ())),
            preferred_element_type=jnp.float32,
        ) * 0.125
        e = jnp.where(qb * QBLK + k_iota <= q_idx, jnp.exp(s), 0.0)
        l = l + jnp.sum(e, axis=-1, keepdims=True)
        acc = acc + jnp.dot(e, v, preferred_element_type=jnp.float32)
        outs.append(acc / l)
    o_ref[...] = jnp.concatenate(outs, axis=1)


def _attn_call(qkv):
    return pl.pallas_call(
        _attn_body,
        grid=(NH // 2, S // QBLK),
        in_specs=[
            pl.BlockSpec((QBLK, 2 * DH), lambda h, i: (i, h)),
            pl.BlockSpec((S, 2 * DH), lambda h, i: (0, NH // 2 + h)),
            pl.BlockSpec((S, 2 * DH), lambda h, i: (0, NH + h)),
        ],
        out_specs=pl.BlockSpec((QBLK, 2 * DH), lambda h, i: (i, h)),
        out_shape=jax.ShapeDtypeStruct((S, H), jnp.float32),
    )(qkv, qkv, qkv)


def _post_body(ctx_ref, x_ref, ow_ref, ob_ref, nw_ref, nb_ref, gw_ref,
               res_ref, hm_ref, lg_ref):
    attn_out = (
        jnp.dot(ctx_ref[...], ow_ref[...], preferred_element_type=jnp.float32)
        + ob_ref[...]
    )
    res = x_ref[...] + attn_out
    res_ref[...] = res
    hm = _ln(res, nw_ref[...], nb_ref[...])
    hm_ref[...] = hm
    lg_ref[...] = jnp.dot(hm, gw_ref[...], preferred_element_type=jnp.float32)


def _post_call(ctx, x, ow, ob, nw, nb, gwp):
    return pl.pallas_call(
        _post_body,
        grid=(S // TBLK,),
        in_specs=[
            pl.BlockSpec((TBLK, H), lambda i: (i, 0)),
            pl.BlockSpec((TBLK, H), lambda i: (i, 0)),
            pl.BlockSpec((H, H), lambda i: (0, 0)),
            pl.BlockSpec((1, H), lambda i: (0, 0)),
            pl.BlockSpec((1, H), lambda i: (0, 0)),
            pl.BlockSpec((1, H), lambda i: (0, 0)),
            pl.BlockSpec((H, 128), lambda i: (0, 0)),
        ],
        out_specs=[
            pl.BlockSpec((TBLK, H), lambda i: (i, 0)),
            pl.BlockSpec((TBLK, H), lambda i: (i, 0)),
            pl.BlockSpec((TBLK, 128), lambda i: (i, 0)),
        ],
        out_shape=[
            jax.ShapeDtypeStruct((S, H), jnp.float32),
            jax.ShapeDtypeStruct((S, H), jnp.float32),
            jax.ShapeDtypeStruct((S, 128), jnp.float32),
        ],
    )(ctx, x, ow, ob, nw, nb, gwp)


def _route_body(lg_ref, g1_ref, cidx_ref, didx_ref, slotk_ref):
    lane = lax.broadcasted_iota(jnp.int32, (TBLK, 128), 1)
    r = lax.broadcasted_iota(jnp.int32, (TBLK, TBLK), 0)
    c = lax.broadcasted_iota(jnp.int32, (TBLK, TBLK), 1)
    tri = (c < r).astype(jnp.float32)
    counts = jnp.zeros((1, 128), jnp.float32)
    for b in range(S // TBLK):
        sl = pl.ds(b * TBLK, TBLK)
        lg = jnp.where(lane < E, lg_ref[sl, :], jnp.float32(-1e30))
        m = jnp.max(lg, axis=1, keepdims=True)
        ex = jnp.exp(lg - m)
        g = ex / jnp.sum(ex, axis=1, keepdims=True)
        gm = jnp.max(g, axis=1, keepdims=True)
        is_max = jnp.logical_and(g == gm, lane < E)
        eidx = jnp.min(jnp.where(is_max, lane, 128), axis=1, keepdims=True)
        mask1 = (lane == eidx).astype(jnp.float32)
        loc_excl = (
            jnp.dot(tri, mask1, preferred_element_type=jnp.float32) + counts
        )
        counts = counts + jnp.sum(mask1, axis=0, keepdims=True)
        loc1 = jnp.sum(loc_excl * mask1, axis=1, keepdims=True)
        keep = loc1 < jnp.float32(CAP)
        g1 = jnp.sum(g * mask1, axis=1, keepdims=True)
        g1_ref[sl, :] = jnp.where(keep, g1, 0.0)
        slot = eidx * CAP + loc1.astype(jnp.int32)
        cidx_ref[sl, :] = jnp.where(keep, slot, 0)
        slotk_ref[sl, :] = jnp.where(keep, slot, -1)
    slotk = slotk_ref[...]
    for sb in range(S // TBLK):
        s_iota = sb * TBLK + lax.broadcasted_iota(jnp.int32, (S, TBLK), 1)
        eq = slotk == s_iota
        t_col = lax.broadcasted_iota(jnp.int32, (S, TBLK), 0)
        inv = jnp.sum(jnp.where(eq, t_col, 0), axis=0, keepdims=True)
        didx_ref[:, pl.ds(sb * TBLK, TBLK)] = inv


def _route_call(logits):
    return pl.pallas_call(
        _route_body,
        grid=(1,),
        in_specs=[pl.BlockSpec((S, 128), lambda i: (0, 0))],
        out_specs=[
            pl.BlockSpec((S, 1), lambda i: (0, 0)),
            pl.BlockSpec((S, 1), lambda i: (0, 0)),
            pl.BlockSpec((1, S), lambda i: (0, 0)),
        ],
        out_shape=[
            jax.ShapeDtypeStruct((S, 1), jnp.float32),
            jax.ShapeDtypeStruct((S, 1), jnp.int32),
            jax.ShapeDtypeStruct((1, S), jnp.int32),
        ],
        scratch_shapes=[pltpu.VMEM((S, 1), jnp.int32)],
    )(logits)


def _sc_gather(table, idx):
    info = plsc.get_sparse_core_info()
    nw = info.num_cores * info.num_subcores
    b = idx.shape[0]
    d = table.shape[1]
    b_per_w = b // nw
    mesh = plsc.VectorSubcoreMesh(core_axis_name="c", subcore_axis_name="s")

    @functools.partial(
        pl.kernel,
        mesh=mesh,
        out_type=jax.ShapeDtypeStruct((b, d), jnp.float32),
        scratch_types=[
            pltpu.VMEM((b_per_w,), jnp.int32),
            pltpu.VMEM((b_per_w, d), jnp.float32),
            pltpu.SemaphoreType.DMA,
        ],
    )
    def k(table_hbm, idx_hbm, out_hbm, idx_v, rows_v, sem):
        wid = lax.axis_index("s") * info.num_cores + lax.axis_index("c")
        base = wid * b_per_w
        pltpu.sync_copy(idx_hbm.at[pl.ds(base, b_per_w)], idx_v)
        pltpu.async_copy(table_hbm.at[idx_v], rows_v, sem).wait()
        pltpu.sync_copy(rows_v, out_hbm.at[pl.ds(base, b_per_w)])

    return k(table, idx)


def _expert_body(d_ref, w1_ref, b1_ref, w2_ref, b2_ref, o_ref):
    f = pl.program_id(1)
    x = d_ref[0]
    h = (
        jnp.dot(x, w1_ref[0], preferred_element_type=jnp.float32) + b1_ref[0]
    )
    h = jax.nn.gelu(h)
    part = jnp.dot(h, w2_ref[0], preferred_element_type=jnp.float32)

    @pl.when(f == 0)
    def _():
        o_ref[0] = part + b2_ref[0]

    @pl.when(f != 0)
    def _():
        o_ref[0] = o_ref[0] + part


def _expert_call(disp, w1, b1, w2, b2):
    return pl.pallas_call(
        _expert_body,
        grid=(E, FF // FBLK),
        in_specs=[
            pl.BlockSpec((1, CAP, H), lambda e, f: (e, 0, 0)),
            pl.BlockSpec((1, H, FBLK), lambda e, f: (e, 0, f)),
            pl.BlockSpec((1, 1, FBLK), lambda e, f: (e, 0, f)),
            pl.BlockSpec((1, FBLK, H), lambda e, f: (e, f, 0)),
            pl.BlockSpec((1, 1, H), lambda e, f: (e, 0, 0)),
        ],
        out_specs=pl.BlockSpec((1, CAP, H), lambda e, f: (e, 0, 0)),
        out_shape=jax.ShapeDtypeStruct((E, CAP, H), jnp.float32),
    )(disp, w1, b1, w2, b2)


def _combine_body(res_ref, rows_ref, g1_ref, o_ref):
    o_ref[...] = res_ref[...] + g1_ref[...] * rows_ref[...]


def _combine_call(res, rows, g1):
    return pl.pallas_call(
        _combine_body,
        grid=(S // TBLK,),
        in_specs=[
            pl.BlockSpec((TBLK, H), lambda i: (i, 0)),
            pl.BlockSpec((TBLK, H), lambda i: (i, 0)),
            pl.BlockSpec((TBLK, 1), lambda i: (i, 0)),
        ],
        out_specs=pl.BlockSpec((TBLK, H), lambda i: (i, 0)),
        out_shape=jax.ShapeDtypeStruct((S, H), jnp.float32),
    )(res, rows, g1)


def kernel(input, qkv_w, qkv_b, attn_ow, attn_ob, norm_w, norm_b,
           attn_nw, attn_nb, gate_w, inter_w, inter_b, output_w, output_b):
    x = input[0]

    qkv = _qkv_call(
        x, norm_w.reshape(1, H), norm_b.reshape(1, H), qkv_w,
        qkv_b.reshape(1, 3 * H),
    )
    ctx = _attn_call(qkv)

    gwp = jnp.pad(gate_w, ((0, 0), (0, 128 - E)))
    residual, hm, logits = _post_call(
        ctx, x, attn_ow, attn_ob.reshape(1, H), attn_nw.reshape(1, H),
        attn_nb.reshape(1, H), gwp,
    )

    g1, cidx, didx = _route_call(logits)

    disp = _sc_gather(hm, didx.reshape(S))
    eout = _expert_call(
        disp.reshape(E, CAP, H), inter_w, inter_b.reshape(E, 1, FF),
        output_w, output_b.reshape(E, 1, H),
    )
    rows = _sc_gather(eout.reshape(E * CAP, H), cidx.reshape(S))

    out = _combine_call(residual, rows, g1)
    return out.reshape(1, S, H)


# speedup vs baseline: 1.7866x; 1.0797x over previous
import functools

import jax
import jax.numpy as jnp
from jax import lax
from jax.experimental import pallas as pl
from jax.experimental.pallas import tpu as pltpu
from jax.experimental.pallas import tpu_sc as plsc

S = 2048
H = 1024
NH = 16
DH = 64
E = 8
FF = 4096
CAP = 256
EPS = 1e-6
TBLK = 256
QBLK = 512
FBLK = 2048


def _ln(x, w, b):
    mu = jnp.mean(x, axis=-1, keepdims=True)
    var = jnp.mean((x - mu) ** 2, axis=-1, keepdims=True)
    return (x - mu) / jnp.sqrt(var + EPS) * w + b


def _qkv_body(x_ref, nw_ref, nb_ref, w_ref, b_ref, o_ref):
    h = _ln(x_ref[...], nw_ref[...], nb_ref[...])
    o_ref[...] = (
        jnp.dot(h, w_ref[...], preferred_element_type=jnp.float32) + b_ref[...]
    )


def _qkv_call(x, nw, nb, w, b):
    return pl.pallas_call(
        _qkv_body,
        grid=(3, S // TBLK),
        in_specs=[
            pl.BlockSpec((TBLK, H), lambda j, i: (i, 0)),
            pl.BlockSpec((1, H), lambda j, i: (0, 0)),
            pl.BlockSpec((1, H), lambda j, i: (0, 0)),
            pl.BlockSpec((H, H), lambda j, i: (0, j)),
            pl.BlockSpec((1, H), lambda j, i: (0, j)),
        ],
        out_specs=pl.BlockSpec((TBLK, H), lambda j, i: (i, j)),
        out_shape=jax.ShapeDtypeStruct((S, 3 * H), jnp.float32),
    )(x, nw, nb, w, b)


def _attn_body(q_ref, k_ref, v_ref, o_ref):
    qb = pl.program_id(1)
    q_idx = qb * QBLK + lax.broadcasted_iota(jnp.int32, (QBLK, QBLK), 0)
    k_iota = lax.broadcasted_iota(jnp.int32, (QBLK, QBLK), 1)
    outs = []
    for hh in range(2):
        q = q_ref[:, pl.ds(hh * DH, DH)]

        def body(kb, carry):
            l, acc = carry
            k = k_ref[pl.ds(kb * QBLK, QBLK), pl.ds(hh * DH, DH)]
            v = v_ref[pl.ds(kb * QBLK, QBLK), pl.ds(hh * DH, DH)]
            s = lax.dot_general(
                q, k, (((1,), (1,)), ((), ())),
                preferred_element_type=jnp.float32,
            ) * 0.125
            e = jnp.exp(s)
            l_new = l + jnp.sum(e, axis=-1, keepdims=True)
            acc_new = acc + jnp.dot(
                e, v, preferred_element_type=jnp.float32
            )
            return l_new, acc_new

        init = (
            jnp.zeros((QBLK, 1), jnp.float32),
            jnp.zeros((QBLK, DH), jnp.float32),
        )
        l, acc = lax.fori_loop(0, qb, body, init)
        k = k_ref[pl.ds(qb * QBLK, QBLK), pl.ds(hh * DH, DH)]
        v = v_ref[pl.ds(qb * QBLK, QBLK), pl.ds(hh * DH, DH)]
        s = lax.dot_general(
            q, k, (((1,), (1,)), ((), ())),
            preferred_element_type=jnp.float32,
        ) * 0.125
        e = jnp.where(qb * QBLK + k_iota <= q_idx, jnp.exp(s), 0.0)
        l = l + jnp.sum(e, axis=-1, keepdims=True)
        acc = acc + jnp.dot(e, v, preferred_element_type=jnp.float32)
        outs.append(acc / l)
    o_ref[...] = jnp.concatenate(outs, axis=1)


def _attn_call(qkv):
    return pl.pallas_call(
        _attn_body,
        grid=(NH // 2, S // QBLK),
        in_specs=[
            pl.BlockSpec((QBLK, 2 * DH), lambda h, i: (i, h)),
            pl.BlockSpec((S, 2 * DH), lambda h, i: (0, NH // 2 + h)),
            pl.BlockSpec((S, 2 * DH), lambda h, i: (0, NH + h)),
        ],
        out_specs=pl.BlockSpec((QBLK, 2 * DH), lambda h, i: (i, h)),
        out_shape=jax.ShapeDtypeStruct((S, H), jnp.float32),
    )(qkv, qkv, qkv)


def _post_body(ctx_ref, x_ref, ow_ref, ob_ref, nw_ref, nb_ref, gw_ref,
               res_ref, hm_ref, lg_ref):
    attn_out = (
        jnp.dot(ctx_ref[...], ow_ref[...], preferred_element_type=jnp.float32)
        + ob_ref[...]
    )
    res = x_ref[...] + attn_out
    res_ref[...] = res
    hm = _ln(res, nw_ref[...], nb_ref[...])
    hm_ref[...] = hm
    lg_ref[...] = jnp.dot(hm, gw_ref[...], preferred_element_type=jnp.float32)


def _post_call(ctx, x, ow, ob, nw, nb, gwp):
    return pl.pallas_call(
        _post_body,
        grid=(S // TBLK,),
        in_specs=[
            pl.BlockSpec((TBLK, H), lambda i: (i, 0)),
            pl.BlockSpec((TBLK, H), lambda i: (i, 0)),
            pl.BlockSpec((H, H), lambda i: (0, 0)),
            pl.BlockSpec((1, H), lambda i: (0, 0)),
            pl.BlockSpec((1, H), lambda i: (0, 0)),
            pl.BlockSpec((1, H), lambda i: (0, 0)),
            pl.BlockSpec((H, 128), lambda i: (0, 0)),
        ],
        out_specs=[
            pl.BlockSpec((TBLK, H), lambda i: (i, 0)),
            pl.BlockSpec((TBLK, H), lambda i: (i, 0)),
            pl.BlockSpec((TBLK, 128), lambda i: (i, 0)),
        ],
        out_shape=[
            jax.ShapeDtypeStruct((S, H), jnp.float32),
            jax.ShapeDtypeStruct((S, H), jnp.float32),
            jax.ShapeDtypeStruct((S, 128), jnp.float32),
        ],
    )(ctx, x, ow, ob, nw, nb, gwp)


def _route_body(lg_ref, g1_ref, cidx_ref, didx_ref, slotk_ref):
    lane = lax.broadcasted_iota(jnp.int32, (TBLK, 128), 1)
    r = lax.broadcasted_iota(jnp.int32, (TBLK, TBLK), 0)
    c = lax.broadcasted_iota(jnp.int32, (TBLK, TBLK), 1)
    tri = (c < r).astype(jnp.float32)
    counts = jnp.zeros((1, 128), jnp.float32)
    for b in range(S // TBLK):
        sl = pl.ds(b * TBLK, TBLK)
        lg = jnp.where(lane < E, lg_ref[sl, :], jnp.float32(-1e30))
        m = jnp.max(lg, axis=1, keepdims=True)
        ex = jnp.exp(lg - m)
        g = ex / jnp.sum(ex, axis=1, keepdims=True)
        gm = jnp.max(g, axis=1, keepdims=True)
        is_max = jnp.logical_and(g == gm, lane < E)
        eidx = jnp.min(jnp.where(is_max, lane, 128), axis=1, keepdims=True)
        mask1 = (lane == eidx).astype(jnp.float32)
        loc_excl = (
            jnp.dot(tri, mask1, preferred_element_type=jnp.float32) + counts
        )
        counts = counts + jnp.sum(mask1, axis=0, keepdims=True)
        loc1 = jnp.sum(loc_excl * mask1, axis=1, keepdims=True)
        keep = loc1 < jnp.float32(CAP)
        g1 = jnp.sum(g * mask1, axis=1, keepdims=True)
        g1_ref[sl, :] = jnp.where(keep, g1, 0.0)
        slot = eidx * CAP + loc1.astype(jnp.int32)
        cidx_ref[sl, :] = jnp.where(keep, slot, 0)
        slotk_ref[sl, :] = jnp.where(keep, slot, -1)
    slotk = slotk_ref[...]
    for sb in range(S // TBLK):
        s_iota = sb * TBLK + lax.broadcasted_iota(jnp.int32, (S, TBLK), 1)
        eq = slotk == s_iota
        t_col = lax.broadcasted_iota(jnp.int32, (S, TBLK), 0)
        inv = jnp.sum(jnp.where(eq, t_col, 0), axis=0, keepdims=True)
        didx_ref[:, pl.ds(sb * TBLK, TBLK)] = inv


def _route_call(logits):
    return pl.pallas_call(
        _route_body,
        grid=(1,),
        in_specs=[pl.BlockSpec((S, 128), lambda i: (0, 0))],
        out_specs=[
            pl.BlockSpec((S, 1), lambda i: (0, 0)),
            pl.BlockSpec((S, 1), lambda i: (0, 0)),
            pl.BlockSpec((1, S), lambda i: (0, 0)),
        ],
        out_shape=[
            jax.ShapeDtypeStruct((S, 1), jnp.float32),
            jax.ShapeDtypeStruct((S, 1), jnp.int32),
            jax.ShapeDtypeStruct((1, S), jnp.int32),
        ],
        scratch_shapes=[pltpu.VMEM((S, 1), jnp.int32)],
    )(logits)


def _sc_gather(table, idx):
    info = plsc.get_sparse_core_info()
    nw = info.num_cores * info.num_subcores
    b = idx.shape[0]
    d = table.shape[1]
    b_per_w = b // nw
    mesh = plsc.VectorSubcoreMesh(core_axis_name="c", subcore_axis_name="s")

    @functools.partial(
        pl.kernel,
        mesh=mesh,
        out_type=jax.ShapeDtypeStruct((b, d), jnp.float32),
        scratch_types=[
            pltpu.VMEM((b_per_w,), jnp.int32),
            pltpu.VMEM((b_per_w, d), jnp.float32),
            pltpu.SemaphoreType.DMA,
        ],
    )
    def k(table_hbm, idx_hbm, out_hbm, idx_v, rows_v, sem):
        wid = lax.axis_index("s") * info.num_cores + lax.axis_index("c")
        base = wid * b_per_w
        pltpu.sync_copy(idx_hbm.at[pl.ds(base, b_per_w)], idx_v)
        pltpu.async_copy(table_hbm.at[idx_v], rows_v, sem).wait()
        pltpu.sync_copy(rows_v, out_hbm.at[pl.ds(base, b_per_w)])

    return k(table, idx)


def _expert_body(d_ref, w1_ref, b1_ref, w2_ref, b2_ref, o_ref):
    f = pl.program_id(1)
    x = d_ref[0]
    h = (
        jnp.dot(x, w1_ref[0], preferred_element_type=jnp.float32) + b1_ref[0]
    )
    h = jax.nn.gelu(h)
    part = jnp.dot(h, w2_ref[0], preferred_element_type=jnp.float32)

    @pl.when(f == 0)
    def _():
        o_ref[0] = part + b2_ref[0]

    @pl.when(f != 0)
    def _():
        o_ref[0] = o_ref[0] + part


def _expert_call(disp, w1, b1, w2, b2):
    return pl.pallas_call(
        _expert_body,
        grid=(E, FF // FBLK),
        in_specs=[
            pl.BlockSpec((1, CAP, H), lambda e, f: (e, 0, 0)),
            pl.BlockSpec((1, H, FBLK), lambda e, f: (e, 0, f)),
            pl.BlockSpec((1, 1, FBLK), lambda e, f: (e, 0, f)),
            pl.BlockSpec((1, FBLK, H), lambda e, f: (e, f, 0)),
            pl.BlockSpec((1, 1, H), lambda e, f: (e, 0, 0)),
        ],
        out_specs=pl.BlockSpec((1, CAP, H), lambda e, f: (e, 0, 0)),
        out_shape=jax.ShapeDtypeStruct((E, CAP, H), jnp.float32),
    )(disp, w1, b1, w2, b2)


def _combine_body(res_ref, rows_ref, g1_ref, o_ref):
    o_ref[...] = res_ref[...] + g1_ref[...] * rows_ref[...]


def _combine_call(res, rows, g1):
    return pl.pallas_call(
        _combine_body,
        grid=(S // TBLK,),
        in_specs=[
            pl.BlockSpec((TBLK, H), lambda i: (i, 0)),
            pl.BlockSpec((TBLK, H), lambda i: (i, 0)),
            pl.BlockSpec((TBLK, 1), lambda i: (i, 0)),
        ],
        out_specs=pl.BlockSpec((TBLK, H), lambda i: (i, 0)),
        out_shape=jax.ShapeDtypeStruct((S, H), jnp.float32),
    )(res, rows, g1)


def kernel(input, qkv_w, qkv_b, attn_ow, attn_ob, norm_w, norm_b,
           attn_nw, attn_nb, gate_w, inter_w, inter_b, output_w, output_b):
    x = input[0]

    qkv = _qkv_call(
        x, norm_w.reshape(1, H), norm_b.reshape(1, H), qkv_w,
        qkv_b.reshape(1, 3 * H),
    )
    ctx = _attn_call(qkv)

    gwp = jnp.pad(gate_w, ((0, 0), (0, 128 - E)))
    residual, hm, logits = _post_call(
        ctx, x, attn_ow, attn_ob.reshape(1, H), attn_nw.reshape(1, H),
        attn_nb.reshape(1, H), gwp,
    )

    g1, cidx, didx = _route_call(logits)

    disp = _sc_gather(hm, didx.reshape(S))
    eout = _expert_call(
        disp.reshape(E, CAP, H), inter_w, inter_b.reshape(E, 1, FF),
        output_w, output_b.reshape(E, 1, H),
    )
    rows = _sc_gather(eout.reshape(E * CAP, H), cidx.reshape(S))

    out = _combine_call(residual, rows, g1)
    return out.reshape(1, S, H)


# speedup vs baseline: 1.8916x; 1.0588x over previous
import functools

import jax
import jax.numpy as jnp
from jax import lax
from jax.experimental import pallas as pl
from jax.experimental.pallas import tpu as pltpu
from jax.experimental.pallas import tpu_sc as plsc

S = 2048
H = 1024
NH = 16
DH = 64
E = 8
FF = 4096
CAP = 256
EPS = 1e-6
TBLK = 256
QBLK = 1024
FBLK = 2048


def _ln(x, w, b):
    mu = jnp.mean(x, axis=-1, keepdims=True)
    var = jnp.mean((x - mu) ** 2, axis=-1, keepdims=True)
    return (x - mu) / jnp.sqrt(var + EPS) * w + b


def _qkv_body(x_ref, nw_ref, nb_ref, w_ref, b_ref, o_ref):
    h = _ln(x_ref[...], nw_ref[...], nb_ref[...])
    o_ref[...] = (
        jnp.dot(h, w_ref[...], preferred_element_type=jnp.float32) + b_ref[...]
    )


def _qkv_call(x, nw, nb, w, b):
    return pl.pallas_call(
        _qkv_body,
        grid=(3, S // TBLK),
        in_specs=[
            pl.BlockSpec((TBLK, H), lambda j, i: (i, 0)),
            pl.BlockSpec((1, H), lambda j, i: (0, 0)),
            pl.BlockSpec((1, H), lambda j, i: (0, 0)),
            pl.BlockSpec((H, H), lambda j, i: (0, j)),
            pl.BlockSpec((1, H), lambda j, i: (0, j)),
        ],
        out_specs=pl.BlockSpec((TBLK, H), lambda j, i: (i, j)),
        out_shape=jax.ShapeDtypeStruct((S, 3 * H), jnp.float32),
    )(x, nw, nb, w, b)


def _attn_body(q_ref, k_ref, v_ref, o_ref):
    qb = pl.program_id(1)
    q_idx = qb * QBLK + lax.broadcasted_iota(jnp.int32, (QBLK, QBLK), 0)
    k_iota = lax.broadcasted_iota(jnp.int32, (QBLK, QBLK), 1)
    outs = []
    for hh in range(2):
        q = q_ref[:, pl.ds(hh * DH, DH)]

        def body(kb, carry):
            l, acc = carry
            k = k_ref[pl.ds(kb * QBLK, QBLK), pl.ds(hh * DH, DH)]
            v = v_ref[pl.ds(kb * QBLK, QBLK), pl.ds(hh * DH, DH)]
            s = lax.dot_general(
                q, k, (((1,), (1,)), ((), ())),
                preferred_element_type=jnp.float32,
            ) * 0.125
            e = jnp.exp(s)
            l_new = l + jnp.sum(e, axis=-1, keepdims=True)
            acc_new = acc + jnp.dot(
                e, v, preferred_element_type=jnp.float32
            )
            return l_new, acc_new

        init = (
            jnp.zeros((QBLK, 1), jnp.float32),
            jnp.zeros((QBLK, DH), jnp.float32),
        )
        l, acc = lax.fori_loop(0, qb, body, init)
        k = k_ref[pl.ds(qb * QBLK, QBLK), pl.ds(hh * DH, DH)]
        v = v_ref[pl.ds(qb * QBLK, QBLK), pl.ds(hh * DH, DH)]
        s = lax.dot_general(
            q, k, (((1,), (1,)), ((), ())),
            preferred_element_type=jnp.float32,
        ) * 0.125
        e = jnp.where(qb * QBLK + k_iota <= q_idx, jnp.exp(s), 0.0)
        l = l + jnp.sum(e, axis=-1, keepdims=True)
        acc = acc + jnp.dot(e, v, preferred_element_type=jnp.float32)
        outs.append(acc / l)
    o_ref[...] = jnp.concatenate(outs, axis=1)


def _attn_call(qkv):
    return pl.pallas_call(
        _attn_body,
        grid=(NH // 2, S // QBLK),
        in_specs=[
            pl.BlockSpec((QBLK, 2 * DH), lambda h, i: (i, h)),
            pl.BlockSpec((S, 2 * DH), lambda h, i: (0, NH // 2 + h)),
            pl.BlockSpec((S, 2 * DH), lambda h, i: (0, NH + h)),
        ],
        out_specs=pl.BlockSpec((QBLK, 2 * DH), lambda h, i: (i, h)),
        out_shape=jax.ShapeDtypeStruct((S, H), jnp.float32),
    )(qkv, qkv, qkv)


def _post_body(ctx_ref, x_ref, ow_ref, ob_ref, nw_ref, nb_ref, gw_ref,
               res_ref, hm_ref, lg_ref):
    attn_out = (
        jnp.dot(ctx_ref[...], ow_ref[...], preferred_element_type=jnp.float32)
        + ob_ref[...]
    )
    res = x_ref[...] + attn_out
    res_ref[...] = res
    hm = _ln(res, nw_ref[...], nb_ref[...])
    hm_ref[...] = hm
    lg_ref[...] = jnp.dot(hm, gw_ref[...], preferred_element_type=jnp.float32)


def _post_call(ctx, x, ow, ob, nw, nb, gwp):
    return pl.pallas_call(
        _post_body,
        grid=(S // TBLK,),
        in_specs=[
            pl.BlockSpec((TBLK, H), lambda i: (i, 0)),
            pl.BlockSpec((TBLK, H), lambda i: (i, 0)),
            pl.BlockSpec((H, H), lambda i: (0, 0)),
            pl.BlockSpec((1, H), lambda i: (0, 0)),
            pl.BlockSpec((1, H), lambda i: (0, 0)),
            pl.BlockSpec((1, H), lambda i: (0, 0)),
            pl.BlockSpec((H, 128), lambda i: (0, 0)),
        ],
        out_specs=[
            pl.BlockSpec((TBLK, H), lambda i: (i, 0)),
            pl.BlockSpec((TBLK, H), lambda i: (i, 0)),
            pl.BlockSpec((TBLK, 128), lambda i: (i, 0)),
        ],
        out_shape=[
            jax.ShapeDtypeStruct((S, H), jnp.float32),
            jax.ShapeDtypeStruct((S, H), jnp.float32),
            jax.ShapeDtypeStruct((S, 128), jnp.float32),
        ],
    )(ctx, x, ow, ob, nw, nb, gwp)


def _route_body(lg_ref, g1_ref, cidx_ref, didx_ref, slotk_ref):
    lane = lax.broadcasted_iota(jnp.int32, (TBLK, 128), 1)
    r = lax.broadcasted_iota(jnp.int32, (TBLK, TBLK), 0)
    c = lax.broadcasted_iota(jnp.int32, (TBLK, TBLK), 1)
    tri = (c < r).astype(jnp.float32)
    counts = jnp.zeros((1, 128), jnp.float32)
    for b in range(S // TBLK):
        sl = pl.ds(b * TBLK, TBLK)
        lg = jnp.where(lane < E, lg_ref[sl, :], jnp.float32(-1e30))
        m = jnp.max(lg, axis=1, keepdims=True)
        ex = jnp.exp(lg - m)
        g = ex / jnp.sum(ex, axis=1, keepdims=True)
        gm = jnp.max(g, axis=1, keepdims=True)
        is_max = jnp.logical_and(g == gm, lane < E)
        eidx = jnp.min(jnp.where(is_max, lane, 128), axis=1, keepdims=True)
        mask1 = (lane == eidx).astype(jnp.float32)
        loc_excl = (
            jnp.dot(tri, mask1, preferred_element_type=jnp.float32) + counts
        )
        counts = counts + jnp.sum(mask1, axis=0, keepdims=True)
        loc1 = jnp.sum(loc_excl * mask1, axis=1, keepdims=True)
        keep = loc1 < jnp.float32(CAP)
        g1 = jnp.sum(g * mask1, axis=1, keepdims=True)
        g1_ref[sl, :] = jnp.where(keep, g1, 0.0)
        slot = eidx * CAP + loc1.astype(jnp.int32)
        cidx_ref[sl, :] = jnp.where(keep, slot, 0)
        slotk_ref[sl, :] = jnp.where(keep, slot, -1)
    slotk = slotk_ref[...]
    for sb in range(S // TBLK):
        s_iota = sb * TBLK + lax.broadcasted_iota(jnp.int32, (S, TBLK), 1)
        eq = slotk == s_iota
        t_col = lax.broadcasted_iota(jnp.int32, (S, TBLK), 0)
        inv = jnp.sum(jnp.where(eq, t_col, 0), axis=0, keepdims=True)
        didx_ref[:, pl.ds(sb * TBLK, TBLK)] = inv


def _route_call(logits):
    return pl.pallas_call(
        _route_body,
        grid=(1,),
        in_specs=[pl.BlockSpec((S, 128), lambda i: (0, 0))],
        out_specs=[
            pl.BlockSpec((S, 1), lambda i: (0, 0)),
            pl.BlockSpec((S, 1), lambda i: (0, 0)),
            pl.BlockSpec((1, S), lambda i: (0, 0)),
        ],
        out_shape=[
            jax.ShapeDtypeStruct((S, 1), jnp.float32),
            jax.ShapeDtypeStruct((S, 1), jnp.int32),
            jax.ShapeDtypeStruct((1, S), jnp.int32),
        ],
        scratch_shapes=[pltpu.VMEM((S, 1), jnp.int32)],
    )(logits)


def _sc_gather(table, idx):
    info = plsc.get_sparse_core_info()
    nw = info.num_cores * info.num_subcores
    b = idx.shape[0]
    d = table.shape[1]
    b_per_w = b // nw
    mesh = plsc.VectorSubcoreMesh(core_axis_name="c", subcore_axis_name="s")

    @functools.partial(
        pl.kernel,
        mesh=mesh,
        out_type=jax.ShapeDtypeStruct((b, d), jnp.float32),
        scratch_types=[
            pltpu.VMEM((b_per_w,), jnp.int32),
            pltpu.VMEM((b_per_w, d), jnp.float32),
            pltpu.SemaphoreType.DMA,
        ],
    )
    def k(table_hbm, idx_hbm, out_hbm, idx_v, rows_v, sem):
        wid = lax.axis_index("s") * info.num_cores + lax.axis_index("c")
        base = wid * b_per_w
        pltpu.sync_copy(idx_hbm.at[pl.ds(base, b_per_w)], idx_v)
        pltpu.async_copy(table_hbm.at[idx_v], rows_v, sem).wait()
        pltpu.sync_copy(rows_v, out_hbm.at[pl.ds(base, b_per_w)])

    return k(table, idx)


def _expert_body(d_ref, w1_ref, b1_ref, w2_ref, b2_ref, o_ref):
    f = pl.program_id(1)
    x = d_ref[0]
    h = (
        jnp.dot(x, w1_ref[0], preferred_element_type=jnp.float32) + b1_ref[0]
    )
    h = jax.nn.gelu(h)
    part = jnp.dot(h, w2_ref[0], preferred_element_type=jnp.float32)

    @pl.when(f == 0)
    def _():
        o_ref[0] = part + b2_ref[0]

    @pl.when(f != 0)
    def _():
        o_ref[0] = o_ref[0] + part


def _expert_call(disp, w1, b1, w2, b2):
    return pl.pallas_call(
        _expert_body,
        grid=(E, FF // FBLK),
        in_specs=[
            pl.BlockSpec((1, CAP, H), lambda e, f: (e, 0, 0)),
            pl.BlockSpec((1, H, FBLK), lambda e, f: (e, 0, f)),
            pl.BlockSpec((1, 1, FBLK), lambda e, f: (e, 0, f)),
            pl.BlockSpec((1, FBLK, H), lambda e, f: (e, f, 0)),
            pl.BlockSpec((1, 1, H), lambda e, f: (e, 0, 0)),
        ],
        out_specs=pl.BlockSpec((1, CAP, H), lambda e, f: (e, 0, 0)),
        out_shape=jax.ShapeDtypeStruct((E, CAP, H), jnp.float32),
    )(disp, w1, b1, w2, b2)


def _combine_body(res_ref, rows_ref, g1_ref, o_ref):
    o_ref[...] = res_ref[...] + g1_ref[...] * rows_ref[...]


def _combine_call(res, rows, g1):
    return pl.pallas_call(
        _combine_body,
        grid=(S // TBLK,),
        in_specs=[
            pl.BlockSpec((TBLK, H), lambda i: (i, 0)),
            pl.BlockSpec((TBLK, H), lambda i: (i, 0)),
            pl.BlockSpec((TBLK, 1), lambda i: (i, 0)),
        ],
        out_specs=pl.BlockSpec((TBLK, H), lambda i: (i, 0)),
        out_shape=jax.ShapeDtypeStruct((S, H), jnp.float32),
    )(res, rows, g1)


def kernel(input, qkv_w, qkv_b, attn_ow, attn_ob, norm_w, norm_b,
           attn_nw, attn_nb, gate_w, inter_w, inter_b, output_w, output_b):
    x = input[0]

    qkv = _qkv_call(
        x, norm_w.reshape(1, H), norm_b.reshape(1, H), qkv_w,
        qkv_b.reshape(1, 3 * H),
    )
    ctx = _attn_call(qkv)

    gwp = jnp.pad(gate_w, ((0, 0), (0, 128 - E)))
    residual, hm, logits = _post_call(
        ctx, x, attn_ow, attn_ob.reshape(1, H), attn_nw.reshape(1, H),
        attn_nb.reshape(1, H), gwp,
    )

    g1, cidx, didx = _route_call(logits)

    disp = _sc_gather(hm, didx.reshape(S))
    eout = _expert_call(
        disp.reshape(E, CAP, H), inter_w, inter_b.reshape(E, 1, FF),
        output_w, output_b.reshape(E, 1, H),
    )
    rows = _sc_gather(eout.reshape(E * CAP, H), cidx.reshape(S))

    out = _combine_call(residual, rows, g1)
    return out.reshape(1, S, H)


# speedup vs baseline: 1.9726x; 1.0428x over previous
import functools

import jax
import jax.numpy as jnp
from jax import lax
from jax.experimental import pallas as pl
from jax.experimental.pallas import tpu as pltpu
from jax.experimental.pallas import tpu_sc as plsc

S = 2048
H = 1024
NH = 16
DH = 64
E = 8
FF = 4096
CAP = 256
EPS = 1e-6
TBLK = 512
QBLK = 1024
FBLK = 2048


def _ln(x, w, b):
    mu = jnp.mean(x, axis=-1, keepdims=True)
    var = jnp.mean((x - mu) ** 2, axis=-1, keepdims=True)
    return (x - mu) / jnp.sqrt(var + EPS) * w + b


def _qkv_body(x_ref, nw_ref, nb_ref, w_ref, b_ref, o_ref):
    h = _ln(x_ref[...], nw_ref[...], nb_ref[...])
    o_ref[...] = (
        jnp.dot(h, w_ref[...], preferred_element_type=jnp.float32) + b_ref[...]
    )


def _qkv_call(x, nw, nb, w, b):
    return pl.pallas_call(
        _qkv_body,
        grid=(3, S // TBLK),
        in_specs=[
            pl.BlockSpec((TBLK, H), lambda j, i: (i, 0)),
            pl.BlockSpec((1, H), lambda j, i: (0, 0)),
            pl.BlockSpec((1, H), lambda j, i: (0, 0)),
            pl.BlockSpec((H, H), lambda j, i: (0, j)),
            pl.BlockSpec((1, H), lambda j, i: (0, j)),
        ],
        out_specs=pl.BlockSpec((TBLK, H), lambda j, i: (i, j)),
        out_shape=jax.ShapeDtypeStruct((S, 3 * H), jnp.float32),
    )(x, nw, nb, w, b)


def _attn_body(q_ref, k_ref, v_ref, o_ref):
    qb = pl.program_id(1)
    q_idx = qb * QBLK + lax.broadcasted_iota(jnp.int32, (QBLK, QBLK), 0)
    k_iota = lax.broadcasted_iota(jnp.int32, (QBLK, QBLK), 1)
    outs = []
    for hh in range(2):
        q = q_ref[:, pl.ds(hh * DH, DH)]

        def body(kb, carry):
            l, acc = carry
            k = k_ref[pl.ds(kb * QBLK, QBLK), pl.ds(hh * DH, DH)]
            v = v_ref[pl.ds(kb * QBLK, QBLK), pl.ds(hh * DH, DH)]
            s = lax.dot_general(
                q, k, (((1,), (1,)), ((), ())),
                preferred_element_type=jnp.float32,
            ) * 0.125
            e = jnp.exp(s)
            l_new = l + jnp.sum(e, axis=-1, keepdims=True)
            acc_new = acc + jnp.dot(
                e, v, preferred_element_type=jnp.float32
            )
            return l_new, acc_new

        init = (
            jnp.zeros((QBLK, 1), jnp.float32),
            jnp.zeros((QBLK, DH), jnp.float32),
        )
        l, acc = lax.fori_loop(0, qb, body, init)
        k = k_ref[pl.ds(qb * QBLK, QBLK), pl.ds(hh * DH, DH)]
        v = v_ref[pl.ds(qb * QBLK, QBLK), pl.ds(hh * DH, DH)]
        s = lax.dot_general(
            q, k, (((1,), (1,)), ((), ())),
            preferred_element_type=jnp.float32,
        ) * 0.125
        e = jnp.where(qb * QBLK + k_iota <= q_idx, jnp.exp(s), 0.0)
        l = l + jnp.sum(e, axis=-1, keepdims=True)
        acc = acc + jnp.dot(e, v, preferred_element_type=jnp.float32)
        outs.append(acc / l)
    o_ref[...] = jnp.concatenate(outs, axis=1)


def _attn_call(qkv):
    return pl.pallas_call(
        _attn_body,
        grid=(NH // 2, S // QBLK),
        in_specs=[
            pl.BlockSpec((QBLK, 2 * DH), lambda h, i: (i, h)),
            pl.BlockSpec((S, 2 * DH), lambda h, i: (0, NH // 2 + h)),
            pl.BlockSpec((S, 2 * DH), lambda h, i: (0, NH + h)),
        ],
        out_specs=pl.BlockSpec((QBLK, 2 * DH), lambda h, i: (i, h)),
        out_shape=jax.ShapeDtypeStruct((S, H), jnp.float32),
    )(qkv, qkv, qkv)


def _post_body(ctx_ref, x_ref, ow_ref, ob_ref, nw_ref, nb_ref, gw_ref,
               res_ref, hm_ref, lg_ref):
    attn_out = (
        jnp.dot(ctx_ref[...], ow_ref[...], preferred_element_type=jnp.float32)
        + ob_ref[...]
    )
    res = x_ref[...] + attn_out
    res_ref[...] = res
    hm = _ln(res, nw_ref[...], nb_ref[...])
    hm_ref[...] = hm
    lg_ref[...] = jnp.dot(hm, gw_ref[...], preferred_element_type=jnp.float32)


def _post_call(ctx, x, ow, ob, nw, nb, gwp):
    return pl.pallas_call(
        _post_body,
        grid=(S // TBLK,),
        in_specs=[
            pl.BlockSpec((TBLK, H), lambda i: (i, 0)),
            pl.BlockSpec((TBLK, H), lambda i: (i, 0)),
            pl.BlockSpec((H, H), lambda i: (0, 0)),
            pl.BlockSpec((1, H), lambda i: (0, 0)),
            pl.BlockSpec((1, H), lambda i: (0, 0)),
            pl.BlockSpec((1, H), lambda i: (0, 0)),
            pl.BlockSpec((H, 128), lambda i: (0, 0)),
        ],
        out_specs=[
            pl.BlockSpec((TBLK, H), lambda i: (i, 0)),
            pl.BlockSpec((TBLK, H), lambda i: (i, 0)),
            pl.BlockSpec((TBLK, 128), lambda i: (i, 0)),
        ],
        out_shape=[
            jax.ShapeDtypeStruct((S, H), jnp.float32),
            jax.ShapeDtypeStruct((S, H), jnp.float32),
            jax.ShapeDtypeStruct((S, 128), jnp.float32),
        ],
    )(ctx, x, ow, ob, nw, nb, gwp)


def _route_body(lg_ref, g1_ref, cidx_ref, didx_ref, slotk_ref):
    lane = lax.broadcasted_iota(jnp.int32, (TBLK, 128), 1)
    r = lax.broadcasted_iota(jnp.int32, (TBLK, TBLK), 0)
    c = lax.broadcasted_iota(jnp.int32, (TBLK, TBLK), 1)
    tri = (c < r).astype(jnp.float32)
    counts = jnp.zeros((1, 128), jnp.float32)
    for b in range(S // TBLK):
        sl = pl.ds(b * TBLK, TBLK)
        lg = jnp.where(lane < E, lg_ref[sl, :], jnp.float32(-1e30))
        m = jnp.max(lg, axis=1, keepdims=True)
        ex = jnp.exp(lg - m)
        g = ex / jnp.sum(ex, axis=1, keepdims=True)
        gm = jnp.max(g, axis=1, keepdims=True)
        is_max = jnp.logical_and(g == gm, lane < E)
        eidx = jnp.min(jnp.where(is_max, lane, 128), axis=1, keepdims=True)
        mask1 = (lane == eidx).astype(jnp.float32)
        loc_excl = (
            jnp.dot(tri, mask1, preferred_element_type=jnp.float32) + counts
        )
        counts = counts + jnp.sum(mask1, axis=0, keepdims=True)
        loc1 = jnp.sum(loc_excl * mask1, axis=1, keepdims=True)
        keep = loc1 < jnp.float32(CAP)
        g1 = jnp.sum(g * mask1, axis=1, keepdims=True)
        g1_ref[sl, :] = jnp.where(keep, g1, 0.0)
        slot = eidx * CAP + loc1.astype(jnp.int32)
        cidx_ref[sl, :] = jnp.where(keep, slot, 0)
        slotk_ref[sl, :] = jnp.where(keep, slot, -1)
    slotk = slotk_ref[...]
    for sb in range(S // TBLK):
        s_iota = sb * TBLK + lax.broadcasted_iota(jnp.int32, (S, TBLK), 1)
        eq = slotk == s_iota
        t_col = lax.broadcasted_iota(jnp.int32, (S, TBLK), 0)
        inv = jnp.sum(jnp.where(eq, t_col, 0), axis=0, keepdims=True)
        didx_ref[:, pl.ds(sb * TBLK, TBLK)] = inv


def _route_call(logits):
    return pl.pallas_call(
        _route_body,
        grid=(1,),
        in_specs=[pl.BlockSpec((S, 128), lambda i: (0, 0))],
        out_specs=[
            pl.BlockSpec((S, 1), lambda i: (0, 0)),
            pl.BlockSpec((S, 1), lambda i: (0, 0)),
            pl.BlockSpec((1, S), lambda i: (0, 0)),
        ],
        out_shape=[
            jax.ShapeDtypeStruct((S, 1), jnp.float32),
            jax.ShapeDtypeStruct((S, 1), jnp.int32),
            jax.ShapeDtypeStruct((1, S), jnp.int32),
        ],
        scratch_shapes=[pltpu.VMEM((S, 1), jnp.int32)],
    )(logits)


def _sc_gather(table, idx):
    info = plsc.get_sparse_core_info()
    nw = info.num_cores * info.num_subcores
    b = idx.shape[0]
    d = table.shape[1]
    b_per_w = b // nw
    mesh = plsc.VectorSubcoreMesh(core_axis_name="c", subcore_axis_name="s")

    @functools.partial(
        pl.kernel,
        mesh=mesh,
        out_type=jax.ShapeDtypeStruct((b, d), jnp.float32),
        scratch_types=[
            pltpu.VMEM((b_per_w,), jnp.int32),
            pltpu.VMEM((b_per_w, d), jnp.float32),
            pltpu.SemaphoreType.DMA,
        ],
    )
    def k(table_hbm, idx_hbm, out_hbm, idx_v, rows_v, sem):
        wid = lax.axis_index("s") * info.num_cores + lax.axis_index("c")
        base = wid * b_per_w
        pltpu.sync_copy(idx_hbm.at[pl.ds(base, b_per_w)], idx_v)
        pltpu.async_copy(table_hbm.at[idx_v], rows_v, sem).wait()
        pltpu.sync_copy(rows_v, out_hbm.at[pl.ds(base, b_per_w)])

    return k(table, idx)


def _expert_body(d_ref, w1_ref, b1_ref, w2_ref, b2_ref, o_ref):
    f = pl.program_id(1)
    x = d_ref[0]
    h = (
        jnp.dot(x, w1_ref[0], preferred_element_type=jnp.float32) + b1_ref[0]
    )
    h = jax.nn.gelu(h)
    part = jnp.dot(h, w2_ref[0], preferred_element_type=jnp.float32)

    @pl.when(f == 0)
    def _():
        o_ref[0] = part + b2_ref[0]

    @pl.when(f != 0)
    def _():
        o_ref[0] = o_ref[0] + part


def _expert_call(disp, w1, b1, w2, b2):
    return pl.pallas_call(
        _expert_body,
        grid=(E, FF // FBLK),
        in_specs=[
            pl.BlockSpec((1, CAP, H), lambda e, f: (e, 0, 0)),
            pl.BlockSpec((1, H, FBLK), lambda e, f: (e, 0, f)),
            pl.BlockSpec((1, 1, FBLK), lambda e, f: (e, 0, f)),
            pl.BlockSpec((1, FBLK, H), lambda e, f: (e, f, 0)),
            pl.BlockSpec((1, 1, H), lambda e, f: (e, 0, 0)),
        ],
        out_specs=pl.BlockSpec((1, CAP, H), lambda e, f: (e, 0, 0)),
        out_shape=jax.ShapeDtypeStruct((E, CAP, H), jnp.float32),
    )(disp, w1, b1, w2, b2)


def _combine_body(res_ref, rows_ref, g1_ref, o_ref):
    o_ref[...] = res_ref[...] + g1_ref[...] * rows_ref[...]


def _combine_call(res, rows, g1):
    return pl.pallas_call(
        _combine_body,
        grid=(S // TBLK,),
        in_specs=[
            pl.BlockSpec((TBLK, H), lambda i: (i, 0)),
            pl.BlockSpec((TBLK, H), lambda i: (i, 0)),
            pl.BlockSpec((TBLK, 1), lambda i: (i, 0)),
        ],
        out_specs=pl.BlockSpec((TBLK, H), lambda i: (i, 0)),
        out_shape=jax.ShapeDtypeStruct((S, H), jnp.float32),
    )(res, rows, g1)


def kernel(input, qkv_w, qkv_b, attn_ow, attn_ob, norm_w, norm_b,
           attn_nw, attn_nb, gate_w, inter_w, inter_b, output_w, output_b):
    x = input[0]

    qkv = _qkv_call(
        x, norm_w.reshape(1, H), norm_b.reshape(1, H), qkv_w,
        qkv_b.reshape(1, 3 * H),
    )
    ctx = _attn_call(qkv)

    gwp = jnp.pad(gate_w, ((0, 0), (0, 128 - E)))
    residual, hm, logits = _post_call(
        ctx, x, attn_ow, attn_ob.reshape(1, H), attn_nw.reshape(1, H),
        attn_nb.reshape(1, H), gwp,
    )

    g1, cidx, didx = _route_call(logits)

    disp = _sc_gather(hm, didx.reshape(S))
    eout = _expert_call(
        disp.reshape(E, CAP, H), inter_w, inter_b.reshape(E, 1, FF),
        output_w, output_b.reshape(E, 1, H),
    )
    rows = _sc_gather(eout.reshape(E * CAP, H), cidx.reshape(S))

    out = _combine_call(residual, rows, g1)
    return out.reshape(1, S, H)
